# Initial kernel scaffold; baseline (speedup 1.0000x reference)
import jax
import jax.numpy as jnp
from jax.experimental import pallas as pl


def kernel(x, edge_index, edge_attr, initial_weights, W_ih, W_hh, b_ih, b_hh, mlp_W1, mlp_b1, mlp_W2, mlp_b2):
    raise NotImplementedError("write your pallas kernel here")


# speedup vs baseline: 7.3851x; 7.3851x over previous
import jax
import jax.numpy as jnp
from jax import lax
from jax.experimental import pallas as pl
from jax.experimental.pallas import tpu as pltpu
from jax.experimental.pallas import tpu_sc as plsc

N = 10000
E = 320000
T = 3
D_IN = 128
D_H = 16
FLAT = D_IN * D_H
G4 = 4 * FLAT

NTILES = 32
CHUNK = 128
ROWS_PT = 80
EPAD = NTILES * ROWS_PT * CHUNK
NPAD = 10240
DUMMY = 10200
STRIPE = NPAD // 16

_f32 = jnp.float32
_mesh = plsc.VectorSubcoreMesh(core_axis_name="c", subcore_axis_name="s")
_SC_PARAMS = pltpu.CompilerParams(use_tc_tiling_on_sc=False)


def _deg_body(dst_hbm, zeros_hbm, ones_hbm, out_hbm, idx_v, ones_v, deg_sh):
    c = lax.axis_index("c")
    s = lax.axis_index("s")
    wid = s * 2 + c
    pltpu.sync_copy(dst_hbm.at[pl.ds(wid * ROWS_PT, ROWS_PT)], idx_v)
    pltpu.sync_copy(ones_hbm, ones_v)
    pltpu.sync_copy(zeros_hbm.at[pl.ds(s * STRIPE, STRIPE)],
                    deg_sh.at[pl.ds(s * STRIPE, STRIPE)])
    plsc.subcore_barrier()

    @pl.loop(0, ROWS_PT)
    def _(j):
        pltpu.sync_copy(ones_v, deg_sh.at[idx_v.at[j]], add=True)

    plsc.subcore_barrier()
    pltpu.sync_copy(deg_sh.at[pl.ds(s * STRIPE, STRIPE)],
                    out_hbm.at[c, pl.ds(s * STRIPE, STRIPE)])


def _deg_call(dst2d, zeros_n16, ones_r):
    k = pl.kernel(
        _deg_body,
        out_type=jax.ShapeDtypeStruct((2, NPAD, D_H), _f32),
        mesh=_mesh,
        scratch_types=[
            pltpu.VMEM((ROWS_PT, CHUNK), jnp.int32),
            pltpu.VMEM((CHUNK, D_H), _f32),
            pltpu.VMEM_SHARED((NPAD, D_H), _f32),
        ],
        compiler_params=_SC_PARAMS,
    )
    return k(dst2d, zeros_n16, ones_r)


def _agg_body(y_hbm, src_hbm, dst_hbm, zeros_hbm, out_hbm,
              isrc_v, idst_v, rows_v, acc_sh):
    c = lax.axis_index("c")
    s = lax.axis_index("s")
    wid = s * 2 + c
    pltpu.sync_copy(src_hbm.at[pl.ds(wid * ROWS_PT, ROWS_PT)], isrc_v)
    pltpu.sync_copy(dst_hbm.at[pl.ds(wid * ROWS_PT, ROWS_PT)], idst_v)
    pltpu.sync_copy(zeros_hbm.at[pl.ds(s * STRIPE, STRIPE)],
                    acc_sh.at[pl.ds(s * STRIPE, STRIPE)])
    plsc.subcore_barrier()

    @pl.loop(0, ROWS_PT)
    def _(j):
        pltpu.sync_copy(y_hbm.at[isrc_v.at[j]], rows_v)
        pltpu.sync_copy(rows_v, acc_sh.at[idst_v.at[j]], add=True)

    plsc.subcore_barrier()
    pltpu.sync_copy(acc_sh.at[pl.ds(s * STRIPE, STRIPE)],
                    out_hbm.at[c, pl.ds(s * STRIPE, STRIPE)])


def _agg_call(y, src2d, dst2d, zeros_n16):
    k = pl.kernel(
        _agg_body,
        out_type=jax.ShapeDtypeStruct((2, NPAD, D_H), _f32),
        mesh=_mesh,
        scratch_types=[
            pltpu.VMEM((ROWS_PT, CHUNK), jnp.int32),
            pltpu.VMEM((ROWS_PT, CHUNK), jnp.int32),
            pltpu.VMEM((CHUNK, D_H), _f32),
            pltpu.VMEM_SHARED((NPAD, D_H), _f32),
        ],
        compiler_params=_SC_PARAMS,
    )
    return k(y, src2d, dst2d, zeros_n16)


def _gath_body(emb_hbm, src_hbm, dst_hbm, gs_hbm, gd_hbm,
               isrc_v, idst_v, bs_v, bd_v):
    c = lax.axis_index("c")
    s = lax.axis_index("s")
    wid = s * 2 + c
    pltpu.sync_copy(src_hbm.at[pl.ds(wid * ROWS_PT, ROWS_PT)], isrc_v)
    pltpu.sync_copy(dst_hbm.at[pl.ds(wid * ROWS_PT, ROWS_PT)], idst_v)

    @pl.loop(0, ROWS_PT)
    def _(j):
        base = (wid * ROWS_PT + j) * CHUNK
        pltpu.sync_copy(emb_hbm.at[isrc_v.at[j]], bs_v)
        pltpu.sync_copy(bs_v, gs_hbm.at[pl.ds(base, CHUNK)])
        pltpu.sync_copy(emb_hbm.at[idst_v.at[j]], bd_v)
        pltpu.sync_copy(bd_v, gd_hbm.at[pl.ds(base, CHUNK)])


def _gath_call(emb, src2d, dst2d):
    k = pl.kernel(
        _gath_body,
        out_type=(jax.ShapeDtypeStruct((EPAD, D_H), _f32),
                  jax.ShapeDtypeStruct((EPAD, D_H), _f32)),
        mesh=_mesh,
        scratch_types=[
            pltpu.VMEM((ROWS_PT, CHUNK), jnp.int32),
            pltpu.VMEM((ROWS_PT, CHUNK), jnp.int32),
            pltpu.VMEM((CHUNK, D_H), _f32),
            pltpu.VMEM((CHUNK, D_H), _f32),
        ],
        compiler_params=_SC_PARAMS,
    )
    return k(emb, src2d, dst2d)


_BR = 512
_NB = G4 // _BR


def _matvec(w, v):
    return lax.dot_general(w, v, (((1,), (0,)), ((), ())),
                           preferred_element_type=_f32)


def _lstm_body(wih_ref, whh_ref, cur0_ref, bih_ref, bhh_ref, hout_ref,
               gates_ref, h_ref, c_ref):
    t = pl.program_id(0)
    nb = pl.program_id(1)

    @pl.when(jnp.logical_and(t == 0, nb == 0))
    def _():
        h_ref[...] = jnp.zeros_like(h_ref)
        c_ref[...] = jnp.zeros_like(c_ref)

    h_cur = h_ref[...]
    v_ih = jnp.where(t == 0, cur0_ref[...], h_cur)
    g = _matvec(wih_ref[...], v_ih) + _matvec(whh_ref[...], h_cur)
    gates_ref[pl.ds(nb * _BR, _BR), :] = g

    @pl.when(nb == _NB - 1)
    def _():
        gf = gates_ref[...] + bih_ref[...] + bhh_ref[...]
        i = jax.nn.sigmoid(gf[0:FLAT])
        f = jax.nn.sigmoid(gf[FLAT:2 * FLAT])
        gg = jnp.tanh(gf[2 * FLAT:3 * FLAT])
        o = jax.nn.sigmoid(gf[3 * FLAT:4 * FLAT])
        c_new = f * c_ref[...] + i * gg
        h_new = o * jnp.tanh(c_new)
        c_ref[...] = c_new
        h_ref[...] = h_new

        @pl.when(t == T - 1)
        def _():
            hout_ref[...] = h_new


def _lstm_call(W_ih, W_hh, cur0, b_ih, b_hh):
    return pl.pallas_call(
        _lstm_body,
        grid=(T, _NB),
        in_specs=[
            pl.BlockSpec((_BR, FLAT), lambda t, nb: (nb, 0)),
            pl.BlockSpec((_BR, FLAT), lambda t, nb: (nb, 0)),
            pl.BlockSpec((FLAT, 1), lambda t, nb: (0, 0)),
            pl.BlockSpec((G4, 1), lambda t, nb: (0, 0)),
            pl.BlockSpec((G4, 1), lambda t, nb: (0, 0)),
        ],
        out_specs=pl.BlockSpec((FLAT, 1), lambda t, nb: (0, 0)),
        out_shape=jax.ShapeDtypeStruct((FLAT, 1), _f32),
        scratch_shapes=[
            pltpu.VMEM((G4, 1), _f32),
            pltpu.VMEM((FLAT, 1), _f32),
            pltpu.VMEM((FLAT, 1), _f32),
        ],
        compiler_params=pltpu.CompilerParams(
            dimension_semantics=("arbitrary", "arbitrary")),
    )(W_ih, W_hh, cur0, b_ih, b_hh)


_BN = 2048


def _ydis_body(x_ref, w_ref, da_ref, db_ref, y_ref, dis_ref):
    deg = da_ref[...][:, 0:1] + db_ref[...][:, 0:1] + 1.0
    dis = lax.rsqrt(deg)
    xw = jnp.dot(x_ref[...], w_ref[...], preferred_element_type=_f32)
    dis_ref[...] = dis
    y_ref[...] = dis * xw


def _ydis_call(x2p, w_fin, degA, degB):
    return pl.pallas_call(
        _ydis_body,
        grid=(NPAD // _BN,),
        in_specs=[
            pl.BlockSpec((_BN, D_IN), lambda i: (i, 0)),
            pl.BlockSpec((D_IN, D_H), lambda i: (0, 0)),
            pl.BlockSpec((_BN, D_H), lambda i: (i, 0)),
            pl.BlockSpec((_BN, D_H), lambda i: (i, 0)),
        ],
        out_specs=[
            pl.BlockSpec((_BN, D_H), lambda i: (i, 0)),
            pl.BlockSpec((_BN, 1), lambda i: (i, 0)),
        ],
        out_shape=[
            jax.ShapeDtypeStruct((NPAD, D_H), _f32),
            jax.ShapeDtypeStruct((NPAD, 1), _f32),
        ],
    )(x2p, w_fin, degA, degB)


def _emb_body(sa_ref, sb_ref, y_ref, dis_ref, emb_ref):
    tot = sa_ref[...] + sb_ref[...] + y_ref[...]
    emb_ref[...] = jnp.maximum(dis_ref[...] * tot, 0.0)


def _emb_call(sA, sB, y, dis):
    return pl.pallas_call(
        _emb_body,
        grid=(NPAD // _BN,),
        in_specs=[
            pl.BlockSpec((_BN, D_H), lambda i: (i, 0)),
            pl.BlockSpec((_BN, D_H), lambda i: (i, 0)),
            pl.BlockSpec((_BN, D_H), lambda i: (i, 0)),
            pl.BlockSpec((_BN, 1), lambda i: (i, 0)),
        ],
        out_specs=pl.BlockSpec((_BN, D_H), lambda i: (i, 0)),
        out_shape=jax.ShapeDtypeStruct((NPAD, D_H), _f32),
    )(sA, sB, y, dis)


_BE = 4096


def _mlp_body(gs_ref, gd_ref, at_ref, w1_ref, b1_ref, w2_ref, b2_ref, out_ref):
    w1 = w1_ref[...]
    dot = lambda a, b: jnp.dot(a, b, preferred_element_type=_f32)
    hid = (dot(gs_ref[...], w1[0:D_H]) + dot(gd_ref[...], w1[D_H:2 * D_H])
           + dot(at_ref[...], w1[2 * D_H:3 * D_H]) + b1_ref[...])
    hid = jnp.maximum(hid, 0.0)
    out_ref[...] = dot(hid, w2_ref[...]) + b2_ref[...]


def _mlp_call(gs, gd, attr, w1, b1, w2, b2):
    return pl.pallas_call(
        _mlp_body,
        grid=(EPAD // _BE,),
        in_specs=[
            pl.BlockSpec((_BE, D_H), lambda i: (i, 0)),
            pl.BlockSpec((_BE, D_H), lambda i: (i, 0)),
            pl.BlockSpec((_BE, D_H), lambda i: (i, 0)),
            pl.BlockSpec((3 * D_H, D_H), lambda i: (0, 0)),
            pl.BlockSpec((1, D_H), lambda i: (0, 0)),
            pl.BlockSpec((D_H, 1), lambda i: (0, 0)),
            pl.BlockSpec((1, 1), lambda i: (0, 0)),
        ],
        out_specs=pl.BlockSpec((_BE, 1), lambda i: (i, 0)),
        out_shape=jax.ShapeDtypeStruct((EPAD, 1), _f32),
    )(gs, gd, attr, w1, b1, w2, b2)


def kernel(x, edge_index, edge_attr, initial_weights, W_ih, W_hh, b_ih, b_hh,
           mlp_W1, mlp_b1, mlp_W2, mlp_b2):
    src = edge_index[0]
    dst = edge_index[1]
    pad = jnp.full((EPAD - E,), DUMMY, jnp.int32)
    src2d = jnp.concatenate([src, pad]).reshape(EPAD // CHUNK, CHUNK)
    dst2d = jnp.concatenate([dst, pad]).reshape(EPAD // CHUNK, CHUNK)

    zeros_n16 = jnp.zeros((NPAD, D_H), _f32)
    ones_r = jnp.ones((CHUNK, D_H), _f32)

    deg2 = _deg_call(dst2d, zeros_n16, ones_r)
    h3 = _lstm_call(W_ih, W_hh, initial_weights.reshape(FLAT, 1),
                    b_ih.reshape(G4, 1), b_hh.reshape(G4, 1))
    w_fin = h3.reshape(D_IN, D_H)

    x2p = jnp.pad(x[T - 1], ((0, NPAD - N), (0, 0)))
    y, dis = _ydis_call(x2p, w_fin, deg2[0], deg2[1])

    s2 = _agg_call(y, src2d, dst2d, zeros_n16)
    emb = _emb_call(s2[0], s2[1], y, dis)
    gs, gd = _gath_call(emb, src2d, dst2d)

    attr_p = jnp.pad(edge_attr, ((0, EPAD - E), (0, 0)))
    logits = _mlp_call(gs, gd, attr_p, mlp_W1,
                       mlp_b1.reshape(1, D_H), mlp_W2,
                       mlp_b2.reshape(1, 1))
    return logits.reshape(EPAD)[:E]


# speedup vs baseline: 8.9585x; 1.2130x over previous
import jax
import jax.numpy as jnp
from jax import lax
from jax.experimental import pallas as pl
from jax.experimental.pallas import tpu as pltpu
from jax.experimental.pallas import tpu_sc as plsc

N = 10000
E = 320000
T = 3
D_IN = 128
D_H = 16
FLAT = D_IN * D_H
G4 = 4 * FLAT

NTILES = 32
CHUNK = 128
NCHUNKS = E // CHUNK
BASE_PT = NCHUNKS // NTILES
EXTRA = NCHUNKS - BASE_PT * NTILES
MAXROWS = BASE_PT + 1
NPAD = 10240
STRIPE = NPAD // 16

_f32 = jnp.float32
_bf16 = jnp.bfloat16
_mesh = plsc.VectorSubcoreMesh(core_axis_name="c", subcore_axis_name="s")
_SC_PARAMS = pltpu.CompilerParams(use_tc_tiling_on_sc=False)


def _tile_range(wid):
    start = BASE_PT * wid + jnp.minimum(wid, EXTRA)
    n = BASE_PT + jnp.where(wid < EXTRA, 1, 0)
    sstart = jnp.minimum(start, NCHUNKS - MAXROWS)
    return start, n, sstart, start - sstart


def _deg_body(dst_hbm, zeros_hbm, ones_hbm, out_hbm, idx_v, ones_v, deg_sh):
    c = lax.axis_index("c")
    s = lax.axis_index("s")
    wid = s * 2 + c
    start, n, sstart, delta = _tile_range(wid)
    pltpu.sync_copy(dst_hbm.at[pl.ds(sstart, MAXROWS)], idx_v)
    pltpu.sync_copy(ones_hbm, ones_v)
    pltpu.sync_copy(zeros_hbm.at[pl.ds(s * STRIPE, STRIPE)],
                    deg_sh.at[pl.ds(s * STRIPE, STRIPE)])
    plsc.subcore_barrier()

    @pl.loop(0, MAXROWS)
    def _(j):
        @pl.when(j < n)
        def _():
            pltpu.sync_copy(ones_v, deg_sh.at[idx_v.at[delta + j]], add=True)

    plsc.subcore_barrier()
    pltpu.sync_copy(deg_sh.at[pl.ds(s * STRIPE, STRIPE)],
                    out_hbm.at[c, pl.ds(s * STRIPE, STRIPE)])


def _deg_call(dst2d, zeros_n16, ones_r):
    k = pl.kernel(
        _deg_body,
        out_type=jax.ShapeDtypeStruct((2, NPAD, D_H), _f32),
        mesh=_mesh,
        scratch_types=[
            pltpu.VMEM((MAXROWS, CHUNK), jnp.int32),
            pltpu.VMEM((CHUNK, D_H), _f32),
            pltpu.VMEM_SHARED((NPAD, D_H), _f32),
        ],
        compiler_params=_SC_PARAMS,
    )
    return k(dst2d, zeros_n16, ones_r)


def _agg_body(y_hbm, src_hbm, dst_hbm, zeros_hbm, out_hbm,
              isrc_v, idst_v, rows_v, acc_sh):
    c = lax.axis_index("c")
    s = lax.axis_index("s")
    wid = s * 2 + c
    start, n, sstart, delta = _tile_range(wid)
    pltpu.sync_copy(src_hbm.at[pl.ds(sstart, MAXROWS)], isrc_v)
    pltpu.sync_copy(dst_hbm.at[pl.ds(sstart, MAXROWS)], idst_v)
    pltpu.sync_copy(zeros_hbm.at[pl.ds(s * STRIPE, STRIPE)],
                    acc_sh.at[pl.ds(s * STRIPE, STRIPE)])
    plsc.subcore_barrier()

    @pl.loop(0, MAXROWS)
    def _(j):
        @pl.when(j < n)
        def _():
            pltpu.sync_copy(y_hbm.at[isrc_v.at[delta + j]], rows_v)
            pltpu.sync_copy(rows_v, acc_sh.at[idst_v.at[delta + j]], add=True)

    plsc.subcore_barrier()
    pltpu.sync_copy(acc_sh.at[pl.ds(s * STRIPE, STRIPE)],
                    out_hbm.at[c, pl.ds(s * STRIPE, STRIPE)])


def _agg_call(y, src2d, dst2d, zeros_n16):
    k = pl.kernel(
        _agg_body,
        out_type=jax.ShapeDtypeStruct((2, NPAD, D_H), _f32),
        mesh=_mesh,
        scratch_types=[
            pltpu.VMEM((MAXROWS, CHUNK), jnp.int32),
            pltpu.VMEM((MAXROWS, CHUNK), jnp.int32),
            pltpu.VMEM((CHUNK, D_H), _f32),
            pltpu.VMEM_SHARED((NPAD, D_H), _f32),
        ],
        compiler_params=_SC_PARAMS,
    )
    return k(y, src2d, dst2d, zeros_n16)


def _gath_body(emb_hbm, src_hbm, dst_hbm, gs_hbm, gd_hbm,
               isrc_v, idst_v, bs_v, bd_v):
    c = lax.axis_index("c")
    s = lax.axis_index("s")
    wid = s * 2 + c
    start, n, sstart, delta = _tile_range(wid)
    pltpu.sync_copy(src_hbm.at[pl.ds(sstart, MAXROWS)], isrc_v)
    pltpu.sync_copy(dst_hbm.at[pl.ds(sstart, MAXROWS)], idst_v)

    @pl.loop(0, MAXROWS)
    def _(j):
        @pl.when(j < n)
        def _():
            base = (start + j) * CHUNK
            pltpu.sync_copy(emb_hbm.at[isrc_v.at[delta + j]], bs_v)
            pltpu.sync_copy(bs_v, gs_hbm.at[pl.ds(base, CHUNK)])
            pltpu.sync_copy(emb_hbm.at[idst_v.at[delta + j]], bd_v)
            pltpu.sync_copy(bd_v, gd_hbm.at[pl.ds(base, CHUNK)])


def _gath_call(emb, src2d, dst2d):
    k = pl.kernel(
        _gath_body,
        out_type=(jax.ShapeDtypeStruct((E, D_H), _f32),
                  jax.ShapeDtypeStruct((E, D_H), _f32)),
        mesh=_mesh,
        scratch_types=[
            pltpu.VMEM((MAXROWS, CHUNK), jnp.int32),
            pltpu.VMEM((MAXROWS, CHUNK), jnp.int32),
            pltpu.VMEM((CHUNK, D_H), _f32),
            pltpu.VMEM((CHUNK, D_H), _f32),
        ],
        compiler_params=_SC_PARAMS,
    )
    return k(emb, src2d, dst2d)


_BR = 256
_NB = G4 // _BR


def _matvec(w, v):
    return lax.dot_general(w, v, (((1,), (0,)), ((), ())),
                           preferred_element_type=_f32)


def _lstm_body(wih_ref, whh_ref, cur0_ref, bih_ref, bhh_ref, hout_ref,
               gates_ref, wsum_ref, h_ref, c_ref):
    t = pl.program_id(0)
    nb = pl.program_id(1)
    rows = pl.ds(nb * _BR, _BR)

    @pl.when(jnp.logical_and(t == 0, nb == 0))
    def _():
        h_ref[...] = jnp.zeros_like(h_ref)
        c_ref[...] = jnp.zeros_like(c_ref)

    @pl.when(t == 0)
    def _():
        wih = wih_ref[...]
        whh = whh_ref[...]
        wsum_ref[rows, :] = (wih + whh).astype(_bf16)
        gates_ref[rows, :] = _matvec(wih, cur0_ref[...])

    @pl.when(t > 0)
    def _():
        w = wsum_ref[rows, :]
        hv = h_ref[...].astype(_bf16)
        gates_ref[rows, :] = _matvec(w, hv)

    @pl.when(nb == _NB - 1)
    def _():
        gf = gates_ref[...] + bih_ref[...] + bhh_ref[...]
        i = jax.nn.sigmoid(gf[0:FLAT])
        f = jax.nn.sigmoid(gf[FLAT:2 * FLAT])
        gg = jnp.tanh(gf[2 * FLAT:3 * FLAT])
        o = jax.nn.sigmoid(gf[3 * FLAT:4 * FLAT])
        c_new = f * c_ref[...] + i * gg
        h_new = o * jnp.tanh(c_new)
        c_ref[...] = c_new
        h_ref[...] = h_new

        @pl.when(t == T - 1)
        def _():
            hout_ref[...] = h_new


def _lstm_call(W_ih, W_hh, cur0, b_ih, b_hh):
    wmap = lambda t, nb: (jnp.where(t == 0, nb, _NB - 1), 0)
    return pl.pallas_call(
        _lstm_body,
        grid=(T, _NB),
        in_specs=[
            pl.BlockSpec((_BR, FLAT), wmap),
            pl.BlockSpec((_BR, FLAT), wmap),
            pl.BlockSpec((FLAT, 1), lambda t, nb: (0, 0)),
            pl.BlockSpec((G4, 1), lambda t, nb: (0, 0)),
            pl.BlockSpec((G4, 1), lambda t, nb: (0, 0)),
        ],
        out_specs=pl.BlockSpec((FLAT, 1), lambda t, nb: (0, 0)),
        out_shape=jax.ShapeDtypeStruct((FLAT, 1), _f32),
        scratch_shapes=[
            pltpu.VMEM((G4, 1), _f32),
            pltpu.VMEM((G4, FLAT), _bf16),
            pltpu.VMEM((FLAT, 1), _f32),
            pltpu.VMEM((FLAT, 1), _f32),
        ],
        compiler_params=pltpu.CompilerParams(
            dimension_semantics=("arbitrary", "arbitrary")),
    )(W_ih, W_hh, cur0, b_ih, b_hh)


_BN = 2048


def _ydis_body(x_ref, w_ref, da_ref, db_ref, y_ref, dis_ref):
    deg = da_ref[...][:, 0:1] + db_ref[...][:, 0:1] + 1.0
    dis = lax.rsqrt(deg)
    xw = jnp.dot(x_ref[...], w_ref[...], preferred_element_type=_f32)
    dis_ref[...] = dis
    y_ref[...] = dis * xw


def _ydis_call(x2p, w_fin, degA, degB):
    return pl.pallas_call(
        _ydis_body,
        grid=(NPAD // _BN,),
        in_specs=[
            pl.BlockSpec((_BN, D_IN), lambda i: (i, 0)),
            pl.BlockSpec((D_IN, D_H), lambda i: (0, 0)),
            pl.BlockSpec((_BN, D_H), lambda i: (i, 0)),
            pl.BlockSpec((_BN, D_H), lambda i: (i, 0)),
        ],
        out_specs=[
            pl.BlockSpec((_BN, D_H), lambda i: (i, 0)),
            pl.BlockSpec((_BN, 1), lambda i: (i, 0)),
        ],
        out_shape=[
            jax.ShapeDtypeStruct((NPAD, D_H), _f32),
            jax.ShapeDtypeStruct((NPAD, 1), _f32),
        ],
    )(x2p, w_fin, degA, degB)


def _emb_body(sa_ref, sb_ref, y_ref, dis_ref, emb_ref):
    tot = sa_ref[...] + sb_ref[...] + y_ref[...]
    emb_ref[...] = jnp.maximum(dis_ref[...] * tot, 0.0)


def _emb_call(sA, sB, y, dis):
    return pl.pallas_call(
        _emb_body,
        grid=(NPAD // _BN,),
        in_specs=[
            pl.BlockSpec((_BN, D_H), lambda i: (i, 0)),
            pl.BlockSpec((_BN, D_H), lambda i: (i, 0)),
            pl.BlockSpec((_BN, D_H), lambda i: (i, 0)),
            pl.BlockSpec((_BN, 1), lambda i: (i, 0)),
        ],
        out_specs=pl.BlockSpec((_BN, D_H), lambda i: (i, 0)),
        out_shape=jax.ShapeDtypeStruct((NPAD, D_H), _f32),
    )(sA, sB, y, dis)


_BE = 6144


def _mlp_body(gs_ref, gd_ref, at_ref, w1_ref, b1_ref, w2_ref, b2_ref, out_ref):
    w1 = w1_ref[...]
    dot = lambda a, b: jnp.dot(a, b, preferred_element_type=_f32)
    hid = (dot(gs_ref[...], w1[0:D_H]) + dot(gd_ref[...], w1[D_H:2 * D_H])
           + dot(at_ref[...], w1[2 * D_H:3 * D_H]) + b1_ref[...])
    hid = jnp.maximum(hid, 0.0)
    out_ref[...] = dot(hid, w2_ref[...]) + b2_ref[...]


def _mlp_call(gs, gd, attr, w1, b1, w2, b2):
    return pl.pallas_call(
        _mlp_body,
        grid=(pl.cdiv(E, _BE),),
        in_specs=[
            pl.BlockSpec((_BE, D_H), lambda i: (i, 0)),
            pl.BlockSpec((_BE, D_H), lambda i: (i, 0)),
            pl.BlockSpec((_BE, D_H), lambda i: (i, 0)),
            pl.BlockSpec((3 * D_H, D_H), lambda i: (0, 0)),
            pl.BlockSpec((1, D_H), lambda i: (0, 0)),
            pl.BlockSpec((D_H, 1), lambda i: (0, 0)),
            pl.BlockSpec((1, 1), lambda i: (0, 0)),
        ],
        out_specs=pl.BlockSpec((_BE, 1), lambda i: (i, 0)),
        out_shape=jax.ShapeDtypeStruct((E, 1), _f32),
    )(gs, gd, attr, w1, b1, w2, b2)


def kernel(x, edge_index, edge_attr, initial_weights, W_ih, W_hh, b_ih, b_hh,
           mlp_W1, mlp_b1, mlp_W2, mlp_b2):
    src2d = edge_index[0].reshape(NCHUNKS, CHUNK)
    dst2d = edge_index[1].reshape(NCHUNKS, CHUNK)

    zeros_n16 = jnp.zeros((NPAD, D_H), _f32)
    ones_r = jnp.ones((CHUNK, D_H), _f32)

    deg2 = _deg_call(dst2d, zeros_n16, ones_r)
    h3 = _lstm_call(W_ih, W_hh, initial_weights.reshape(FLAT, 1),
                    b_ih.reshape(G4, 1), b_hh.reshape(G4, 1))
    w_fin = h3.reshape(D_IN, D_H)

    x2p = jnp.pad(x[T - 1], ((0, NPAD - N), (0, 0)))
    y, dis = _ydis_call(x2p, w_fin, deg2[0], deg2[1])

    s2 = _agg_call(y, src2d, dst2d, zeros_n16)
    emb = _emb_call(s2[0], s2[1], y, dis)
    gs, gd = _gath_call(emb, src2d, dst2d)

    logits = _mlp_call(gs, gd, edge_attr, mlp_W1,
                       mlp_b1.reshape(1, D_H), mlp_W2,
                       mlp_b2.reshape(1, 1))
    return logits.reshape(E)


# speedup vs baseline: 10.2974x; 1.1495x over previous
import jax
import jax.numpy as jnp
from jax import lax
from jax.experimental import pallas as pl
from jax.experimental.pallas import tpu as pltpu
from jax.experimental.pallas import tpu_sc as plsc

N = 10000
E = 320000
T = 3
D_IN = 128
D_H = 16
FLAT = D_IN * D_H
G4 = 4 * FLAT

NTILES = 32
CHUNK = 128
NCHUNKS = E // CHUNK
BASE_PT = NCHUNKS // NTILES
EXTRA = NCHUNKS - BASE_PT * NTILES
MAXROWS = BASE_PT + 1
GROUP = 8
NGROUP = (MAXROWS + GROUP - 1) // GROUP
NPAD = 10240
STRIPE = NPAD // 16

_f32 = jnp.float32
_bf16 = jnp.bfloat16
_mesh = plsc.VectorSubcoreMesh(core_axis_name="c", subcore_axis_name="s")
_SC_PARAMS = pltpu.CompilerParams(use_tc_tiling_on_sc=False)


def _tile_range(wid):
    start = BASE_PT * wid + jnp.minimum(wid, EXTRA)
    n = BASE_PT + jnp.where(wid < EXTRA, 1, 0)
    sstart = jnp.minimum(start, NCHUNKS - MAXROWS)
    return start, n, sstart, start - sstart


def _deg_body(dst_hbm, zeros_hbm, ones_hbm, out_hbm, idx_v, ones_v, deg_sh,
              sem):
    c = lax.axis_index("c")
    s = lax.axis_index("s")
    wid = s * 2 + c
    start, n, sstart, delta = _tile_range(wid)
    pltpu.sync_copy(dst_hbm.at[pl.ds(sstart, MAXROWS)], idx_v)
    pltpu.sync_copy(ones_hbm, ones_v)
    pltpu.sync_copy(zeros_hbm.at[pl.ds(s * STRIPE, STRIPE)],
                    deg_sh.at[pl.ds(s * STRIPE, STRIPE)])
    plsc.subcore_barrier()

    @pl.loop(0, NGROUP)
    def _(g):
        for b in range(GROUP):
            @pl.when(g * GROUP + b < n)
            def _(b=b):
                j = g * GROUP + b
                pltpu.async_copy(ones_v, deg_sh.at[idx_v.at[delta + j]],
                                 sem, add=True)
        for b in range(GROUP):
            @pl.when(g * GROUP + b < n)
            def _(b=b):
                j = g * GROUP + b
                pltpu.make_async_copy(ones_v, deg_sh.at[idx_v.at[delta + j]],
                                      sem).wait()

    plsc.subcore_barrier()
    pltpu.sync_copy(deg_sh.at[pl.ds(s * STRIPE, STRIPE)],
                    out_hbm.at[c, pl.ds(s * STRIPE, STRIPE)])


def _deg_call(dst2d, zeros_n16, ones_r):
    k = pl.kernel(
        _deg_body,
        out_type=jax.ShapeDtypeStruct((2, NPAD, D_H), _f32),
        mesh=_mesh,
        scratch_types=[
            pltpu.VMEM((MAXROWS, CHUNK), jnp.int32),
            pltpu.VMEM((CHUNK, D_H), _f32),
            pltpu.VMEM_SHARED((NPAD, D_H), _f32),
            pltpu.SemaphoreType.DMA,
        ],
        compiler_params=_SC_PARAMS,
    )
    return k(dst2d, zeros_n16, ones_r)


def _agg_body(y_hbm, src_hbm, dst_hbm, zeros_hbm, out_hbm,
              isrc_v, idst_v, rows_v, acc_sh, sem_g, sem_s):
    c = lax.axis_index("c")
    s = lax.axis_index("s")
    wid = s * 2 + c
    start, n, sstart, delta = _tile_range(wid)
    pltpu.sync_copy(src_hbm.at[pl.ds(sstart, MAXROWS)], isrc_v)
    pltpu.sync_copy(dst_hbm.at[pl.ds(sstart, MAXROWS)], idst_v)
    pltpu.sync_copy(zeros_hbm.at[pl.ds(s * STRIPE, STRIPE)],
                    acc_sh.at[pl.ds(s * STRIPE, STRIPE)])
    plsc.subcore_barrier()

    @pl.loop(0, NGROUP)
    def _(g):
        for b in range(GROUP):
            @pl.when(g * GROUP + b < n)
            def _(b=b):
                j = g * GROUP + b
                pltpu.async_copy(y_hbm.at[isrc_v.at[delta + j]],
                                 rows_v.at[pl.ds(b * CHUNK, CHUNK)], sem_g)
        for b in range(GROUP):
            @pl.when(g * GROUP + b < n)
            def _(b=b):
                j = g * GROUP + b
                pltpu.make_async_copy(y_hbm.at[isrc_v.at[delta + j]],
                                      rows_v.at[pl.ds(b * CHUNK, CHUNK)],
                                      sem_g).wait()
        for b in range(GROUP):
            @pl.when(g * GROUP + b < n)
            def _(b=b):
                j = g * GROUP + b
                pltpu.async_copy(rows_v.at[pl.ds(b * CHUNK, CHUNK)],
                                 acc_sh.at[idst_v.at[delta + j]], sem_s,
                                 add=True)
        for b in range(GROUP):
            @pl.when(g * GROUP + b < n)
            def _(b=b):
                j = g * GROUP + b
                pltpu.make_async_copy(rows_v.at[pl.ds(b * CHUNK, CHUNK)],
                                      acc_sh.at[idst_v.at[delta + j]],
                                      sem_s).wait()

    plsc.subcore_barrier()
    pltpu.sync_copy(acc_sh.at[pl.ds(s * STRIPE, STRIPE)],
                    out_hbm.at[c, pl.ds(s * STRIPE, STRIPE)])


def _agg_call(y, src2d, dst2d, zeros_n16):
    k = pl.kernel(
        _agg_body,
        out_type=jax.ShapeDtypeStruct((2, NPAD, D_H), _f32),
        mesh=_mesh,
        scratch_types=[
            pltpu.VMEM((MAXROWS, CHUNK), jnp.int32),
            pltpu.VMEM((MAXROWS, CHUNK), jnp.int32),
            pltpu.VMEM((GROUP * CHUNK, D_H), _f32),
            pltpu.VMEM_SHARED((NPAD, D_H), _f32),
            pltpu.SemaphoreType.DMA,
            pltpu.SemaphoreType.DMA,
        ],
        compiler_params=_SC_PARAMS,
    )
    return k(y, src2d, dst2d, zeros_n16)


def _gath_body(emb_hbm, src_hbm, dst_hbm, gs_hbm, gd_hbm,
               isrc_v, idst_v, bs_v, bd_v, sem_g):
    c = lax.axis_index("c")
    s = lax.axis_index("s")
    wid = s * 2 + c
    start, n, sstart, delta = _tile_range(wid)
    pltpu.sync_copy(src_hbm.at[pl.ds(sstart, MAXROWS)], isrc_v)
    pltpu.sync_copy(dst_hbm.at[pl.ds(sstart, MAXROWS)], idst_v)

    @pl.loop(0, NGROUP)
    def _(g):
        for b in range(GROUP):
            @pl.when(g * GROUP + b < n)
            def _(b=b):
                j = g * GROUP + b
                pltpu.async_copy(emb_hbm.at[isrc_v.at[delta + j]],
                                 bs_v.at[pl.ds(b * CHUNK, CHUNK)], sem_g)
                pltpu.async_copy(emb_hbm.at[idst_v.at[delta + j]],
                                 bd_v.at[pl.ds(b * CHUNK, CHUNK)], sem_g)
        for b in range(GROUP):
            @pl.when(g * GROUP + b < n)
            def _(b=b):
                j = g * GROUP + b
                pltpu.make_async_copy(emb_hbm.at[isrc_v.at[delta + j]],
                                      bs_v.at[pl.ds(b * CHUNK, CHUNK)],
                                      sem_g).wait()
                pltpu.make_async_copy(emb_hbm.at[idst_v.at[delta + j]],
                                      bd_v.at[pl.ds(b * CHUNK, CHUNK)],
                                      sem_g).wait()

        @pl.when(g * GROUP + GROUP <= n)
        def _():
            base = (start + g * GROUP) * CHUNK
            pltpu.sync_copy(bs_v, gs_hbm.at[pl.ds(base, GROUP * CHUNK)])
            pltpu.sync_copy(bd_v, gd_hbm.at[pl.ds(base, GROUP * CHUNK)])

        @pl.when(jnp.logical_and(g * GROUP < n, g * GROUP + GROUP > n))
        def _():
            for b in range(GROUP):
                @pl.when(g * GROUP + b < n)
                def _(b=b):
                    j = g * GROUP + b
                    base = (start + j) * CHUNK
                    pltpu.sync_copy(bs_v.at[pl.ds(b * CHUNK, CHUNK)],
                                    gs_hbm.at[pl.ds(base, CHUNK)])
                    pltpu.sync_copy(bd_v.at[pl.ds(b * CHUNK, CHUNK)],
                                    gd_hbm.at[pl.ds(base, CHUNK)])


def _gath_call(emb, src2d, dst2d):
    k = pl.kernel(
        _gath_body,
        out_type=(jax.ShapeDtypeStruct((E, D_H), _f32),
                  jax.ShapeDtypeStruct((E, D_H), _f32)),
        mesh=_mesh,
        scratch_types=[
            pltpu.VMEM((MAXROWS, CHUNK), jnp.int32),
            pltpu.VMEM((MAXROWS, CHUNK), jnp.int32),
            pltpu.VMEM((GROUP * CHUNK, D_H), _f32),
            pltpu.VMEM((GROUP * CHUNK, D_H), _f32),
            pltpu.SemaphoreType.DMA,
        ],
        compiler_params=_SC_PARAMS,
    )
    return k(emb, src2d, dst2d)


_BR = 256
_NB = G4 // _BR


def _matvec(w, v):
    return lax.dot_general(w, v, (((1,), (0,)), ((), ())),
                           preferred_element_type=_f32)


def _lstm_body(wih_ref, whh_ref, cur0_ref, bih_ref, bhh_ref, hout_ref,
               gates_ref, wsum_ref, h_ref, c_ref):
    t = pl.program_id(0)
    nb = pl.program_id(1)
    rows = pl.ds(nb * _BR, _BR)

    @pl.when(jnp.logical_and(t == 0, nb == 0))
    def _():
        h_ref[...] = jnp.zeros_like(h_ref)
        c_ref[...] = jnp.zeros_like(c_ref)

    @pl.when(t == 0)
    def _():
        wih = wih_ref[...]
        whh = whh_ref[...]
        wsum_ref[rows, :] = (wih + whh).astype(_bf16)
        gates_ref[rows, :] = _matvec(wih, cur0_ref[...])

    @pl.when(t > 0)
    def _():
        w = wsum_ref[rows, :]
        hv = h_ref[...].astype(_bf16)
        gates_ref[rows, :] = _matvec(w, hv)

    @pl.when(nb == _NB - 1)
    def _():
        gf = gates_ref[...] + bih_ref[...] + bhh_ref[...]
        i = jax.nn.sigmoid(gf[0:FLAT])
        f = jax.nn.sigmoid(gf[FLAT:2 * FLAT])
        gg = jnp.tanh(gf[2 * FLAT:3 * FLAT])
        o = jax.nn.sigmoid(gf[3 * FLAT:4 * FLAT])
        c_new = f * c_ref[...] + i * gg
        h_new = o * jnp.tanh(c_new)
        c_ref[...] = c_new
        h_ref[...] = h_new

        @pl.when(t == T - 1)
        def _():
            hout_ref[...] = h_new


def _lstm_call(W_ih, W_hh, cur0, b_ih, b_hh):
    wmap = lambda t, nb: (jnp.where(t == 0, nb, _NB - 1), 0)
    return pl.pallas_call(
        _lstm_body,
        grid=(T, _NB),
        in_specs=[
            pl.BlockSpec((_BR, FLAT), wmap),
            pl.BlockSpec((_BR, FLAT), wmap),
            pl.BlockSpec((FLAT, 1), lambda t, nb: (0, 0)),
            pl.BlockSpec((G4, 1), lambda t, nb: (0, 0)),
            pl.BlockSpec((G4, 1), lambda t, nb: (0, 0)),
        ],
        out_specs=pl.BlockSpec((FLAT, 1), lambda t, nb: (0, 0)),
        out_shape=jax.ShapeDtypeStruct((FLAT, 1), _f32),
        scratch_shapes=[
            pltpu.VMEM((G4, 1), _f32),
            pltpu.VMEM((G4, FLAT), _bf16),
            pltpu.VMEM((FLAT, 1), _f32),
            pltpu.VMEM((FLAT, 1), _f32),
        ],
        compiler_params=pltpu.CompilerParams(
            dimension_semantics=("arbitrary", "arbitrary")),
    )(W_ih, W_hh, cur0, b_ih, b_hh)


_BN = 2048


def _ydis_body(x_ref, w_ref, da_ref, db_ref, y_ref, dis_ref):
    deg = da_ref[...][:, 0:1] + db_ref[...][:, 0:1] + 1.0
    dis = lax.rsqrt(deg)
    xw = jnp.dot(x_ref[...], w_ref[...], preferred_element_type=_f32)
    dis_ref[...] = dis
    y_ref[...] = dis * xw


def _ydis_call(x2p, w_fin, degA, degB):
    return pl.pallas_call(
        _ydis_body,
        grid=(NPAD // _BN,),
        in_specs=[
            pl.BlockSpec((_BN, D_IN), lambda i: (i, 0)),
            pl.BlockSpec((D_IN, D_H), lambda i: (0, 0)),
            pl.BlockSpec((_BN, D_H), lambda i: (i, 0)),
            pl.BlockSpec((_BN, D_H), lambda i: (i, 0)),
        ],
        out_specs=[
            pl.BlockSpec((_BN, D_H), lambda i: (i, 0)),
            pl.BlockSpec((_BN, 1), lambda i: (i, 0)),
        ],
        out_shape=[
            jax.ShapeDtypeStruct((NPAD, D_H), _f32),
            jax.ShapeDtypeStruct((NPAD, 1), _f32),
        ],
    )(x2p, w_fin, degA, degB)


def _emb_body(sa_ref, sb_ref, y_ref, dis_ref, emb_ref):
    tot = sa_ref[...] + sb_ref[...] + y_ref[...]
    emb_ref[...] = jnp.maximum(dis_ref[...] * tot, 0.0)


def _emb_call(sA, sB, y, dis):
    return pl.pallas_call(
        _emb_body,
        grid=(NPAD // _BN,),
        in_specs=[
            pl.BlockSpec((_BN, D_H), lambda i: (i, 0)),
            pl.BlockSpec((_BN, D_H), lambda i: (i, 0)),
            pl.BlockSpec((_BN, D_H), lambda i: (i, 0)),
            pl.BlockSpec((_BN, 1), lambda i: (i, 0)),
        ],
        out_specs=pl.BlockSpec((_BN, D_H), lambda i: (i, 0)),
        out_shape=jax.ShapeDtypeStruct((NPAD, D_H), _f32),
    )(sA, sB, y, dis)


_BE = 6144


def _mlp_body(gs_ref, gd_ref, at_ref, w1_ref, b1_ref, w2_ref, b2_ref, out_ref):
    w1 = w1_ref[...]
    dot = lambda a, b: jnp.dot(a, b, preferred_element_type=_f32)
    hid = (dot(gs_ref[...], w1[0:D_H]) + dot(gd_ref[...], w1[D_H:2 * D_H])
           + dot(at_ref[...], w1[2 * D_H:3 * D_H]) + b1_ref[...])
    hid = jnp.maximum(hid, 0.0)
    out_ref[...] = dot(hid, w2_ref[...]) + b2_ref[...]


def _mlp_call(gs, gd, attr, w1, b1, w2, b2):
    return pl.pallas_call(
        _mlp_body,
        grid=(pl.cdiv(E, _BE),),
        in_specs=[
            pl.BlockSpec((_BE, D_H), lambda i: (i, 0)),
            pl.BlockSpec((_BE, D_H), lambda i: (i, 0)),
            pl.BlockSpec((_BE, D_H), lambda i: (i, 0)),
            pl.BlockSpec((3 * D_H, D_H), lambda i: (0, 0)),
            pl.BlockSpec((1, D_H), lambda i: (0, 0)),
            pl.BlockSpec((D_H, 1), lambda i: (0, 0)),
            pl.BlockSpec((1, 1), lambda i: (0, 0)),
        ],
        out_specs=pl.BlockSpec((_BE, 1), lambda i: (i, 0)),
        out_shape=jax.ShapeDtypeStruct((E, 1), _f32),
    )(gs, gd, attr, w1, b1, w2, b2)


def kernel(x, edge_index, edge_attr, initial_weights, W_ih, W_hh, b_ih, b_hh,
           mlp_W1, mlp_b1, mlp_W2, mlp_b2):
    src2d = edge_index[0].reshape(NCHUNKS, CHUNK)
    dst2d = edge_index[1].reshape(NCHUNKS, CHUNK)

    zeros_n16 = jnp.zeros((NPAD, D_H), _f32)
    ones_r = jnp.ones((CHUNK, D_H), _f32)

    deg2 = _deg_call(dst2d, zeros_n16, ones_r)
    h3 = _lstm_call(W_ih, W_hh, initial_weights.reshape(FLAT, 1),
                    b_ih.reshape(G4, 1), b_hh.reshape(G4, 1))
    w_fin = h3.reshape(D_IN, D_H)

    x2p = jnp.pad(x[T - 1], ((0, NPAD - N), (0, 0)))
    y, dis = _ydis_call(x2p, w_fin, deg2[0], deg2[1])

    s2 = _agg_call(y, src2d, dst2d, zeros_n16)
    emb = _emb_call(s2[0], s2[1], y, dis)
    gs, gd = _gath_call(emb, src2d, dst2d)

    logits = _mlp_call(gs, gd, edge_attr, mlp_W1,
                       mlp_b1.reshape(1, D_H), mlp_W2,
                       mlp_b2.reshape(1, 1))
    return logits.reshape(E)


# speedup vs baseline: 10.5242x; 1.0220x over previous
import jax
import jax.numpy as jnp
from jax import lax
from jax.experimental import pallas as pl
from jax.experimental.pallas import tpu as pltpu
from jax.experimental.pallas import tpu_sc as plsc

N = 10000
E = 320000
T = 3
D_IN = 128
D_H = 16
FLAT = D_IN * D_H
G4 = 4 * FLAT

NTILES = 32
CHUNK = 128
NCHUNKS = E // CHUNK
BASE_PT = NCHUNKS // NTILES
EXTRA = NCHUNKS - BASE_PT * NTILES
MAXROWS = BASE_PT + 1
GROUP = 16
NGROUP = (MAXROWS + GROUP - 1) // GROUP
NPAD = 10240
STRIPE = NPAD // 16

_f32 = jnp.float32
_bf16 = jnp.bfloat16
_mesh = plsc.VectorSubcoreMesh(core_axis_name="c", subcore_axis_name="s")
_SC_PARAMS = pltpu.CompilerParams(use_tc_tiling_on_sc=False)


def _tile_range(wid):
    start = BASE_PT * wid + jnp.minimum(wid, EXTRA)
    n = BASE_PT + jnp.where(wid < EXTRA, 1, 0)
    sstart = jnp.minimum(start, NCHUNKS - MAXROWS)
    return start, n, sstart, start - sstart


def _deg_body(eidx_hbm, zeros_hbm, ones_hbm, out_hbm, idx_v, ones_v, deg_sh,
              sem):
    c = lax.axis_index("c")
    s = lax.axis_index("s")
    wid = s * 2 + c
    start, n, sstart, delta = _tile_range(wid)
    pltpu.sync_copy(eidx_hbm.at[1, pl.ds(sstart, MAXROWS)], idx_v)
    pltpu.sync_copy(ones_hbm, ones_v)
    pltpu.sync_copy(zeros_hbm.at[pl.ds(s * STRIPE, STRIPE)],
                    deg_sh.at[pl.ds(s * STRIPE, STRIPE)])
    plsc.subcore_barrier()

    @pl.loop(0, NGROUP)
    def _(g):
        for b in range(GROUP):
            @pl.when(g * GROUP + b < n)
            def _(b=b):
                j = g * GROUP + b
                pltpu.async_copy(ones_v, deg_sh.at[idx_v.at[delta + j]],
                                 sem, add=True)
        for b in range(GROUP):
            @pl.when(g * GROUP + b < n)
            def _(b=b):
                j = g * GROUP + b
                pltpu.make_async_copy(ones_v, deg_sh.at[idx_v.at[delta + j]],
                                      sem).wait()

    plsc.subcore_barrier()
    pltpu.sync_copy(deg_sh.at[pl.ds(s * STRIPE, STRIPE)],
                    out_hbm.at[c, pl.ds(s * STRIPE, STRIPE)])


def _deg_call(eidx3, zeros_n16, ones_r):
    k = pl.kernel(
        _deg_body,
        out_type=jax.ShapeDtypeStruct((2, NPAD, D_H), _f32),
        mesh=_mesh,
        scratch_types=[
            pltpu.VMEM((MAXROWS, CHUNK), jnp.int32),
            pltpu.VMEM((CHUNK, D_H), _f32),
            pltpu.VMEM_SHARED((NPAD, D_H), _f32),
            pltpu.SemaphoreType.DMA,
        ],
        compiler_params=_SC_PARAMS,
    )
    return k(eidx3, zeros_n16, ones_r)


def _agg_body(y_hbm, eidx_hbm, zeros_hbm, out_hbm,
              isrc_v, idst_v, rows_v, acc_sh, sem_g, sem_s):
    c = lax.axis_index("c")
    s = lax.axis_index("s")
    wid = s * 2 + c
    start, n, sstart, delta = _tile_range(wid)
    pltpu.sync_copy(eidx_hbm.at[0, pl.ds(sstart, MAXROWS)], isrc_v)
    pltpu.sync_copy(eidx_hbm.at[1, pl.ds(sstart, MAXROWS)], idst_v)
    pltpu.sync_copy(zeros_hbm.at[pl.ds(s * STRIPE, STRIPE)],
                    acc_sh.at[pl.ds(s * STRIPE, STRIPE)])
    plsc.subcore_barrier()

    @pl.loop(0, NGROUP)
    def _(g):
        for b in range(GROUP):
            @pl.when(g * GROUP + b < n)
            def _(b=b):
                j = g * GROUP + b
                pltpu.async_copy(y_hbm.at[isrc_v.at[delta + j]],
                                 rows_v.at[pl.ds(b * CHUNK, CHUNK)], sem_g)
        for b in range(GROUP):
            @pl.when(g * GROUP + b < n)
            def _(b=b):
                j = g * GROUP + b
                pltpu.make_async_copy(y_hbm.at[isrc_v.at[delta + j]],
                                      rows_v.at[pl.ds(b * CHUNK, CHUNK)],
                                      sem_g).wait()
        for b in range(GROUP):
            @pl.when(g * GROUP + b < n)
            def _(b=b):
                j = g * GROUP + b
                pltpu.async_copy(rows_v.at[pl.ds(b * CHUNK, CHUNK)],
                                 acc_sh.at[idst_v.at[delta + j]], sem_s,
                                 add=True)
        for b in range(GROUP):
            @pl.when(g * GROUP + b < n)
            def _(b=b):
                j = g * GROUP + b
                pltpu.make_async_copy(rows_v.at[pl.ds(b * CHUNK, CHUNK)],
                                      acc_sh.at[idst_v.at[delta + j]],
                                      sem_s).wait()

    plsc.subcore_barrier()
    pltpu.sync_copy(acc_sh.at[pl.ds(s * STRIPE, STRIPE)],
                    out_hbm.at[c, pl.ds(s * STRIPE, STRIPE)])


def _agg_call(y, eidx3, zeros_n16):
    k = pl.kernel(
        _agg_body,
        out_type=jax.ShapeDtypeStruct((2, NPAD, D_H), _f32),
        mesh=_mesh,
        scratch_types=[
            pltpu.VMEM((MAXROWS, CHUNK), jnp.int32),
            pltpu.VMEM((MAXROWS, CHUNK), jnp.int32),
            pltpu.VMEM((GROUP * CHUNK, D_H), _f32),
            pltpu.VMEM_SHARED((NPAD, D_H), _f32),
            pltpu.SemaphoreType.DMA,
            pltpu.SemaphoreType.DMA,
        ],
        compiler_params=_SC_PARAMS,
    )
    return k(y, eidx3, zeros_n16)


def _gath_body(emb_hbm, eidx_hbm, gs_hbm, gd_hbm,
               isrc_v, idst_v, bs_v, bd_v, sem_g):
    c = lax.axis_index("c")
    s = lax.axis_index("s")
    wid = s * 2 + c
    start, n, sstart, delta = _tile_range(wid)
    pltpu.sync_copy(eidx_hbm.at[0, pl.ds(sstart, MAXROWS)], isrc_v)
    pltpu.sync_copy(eidx_hbm.at[1, pl.ds(sstart, MAXROWS)], idst_v)

    @pl.loop(0, NGROUP)
    def _(g):
        for b in range(GROUP):
            @pl.when(g * GROUP + b < n)
            def _(b=b):
                j = g * GROUP + b
                pltpu.async_copy(emb_hbm.at[isrc_v.at[delta + j]],
                                 bs_v.at[pl.ds(b * CHUNK, CHUNK)], sem_g)
                pltpu.async_copy(emb_hbm.at[idst_v.at[delta + j]],
                                 bd_v.at[pl.ds(b * CHUNK, CHUNK)], sem_g)
        for b in range(GROUP):
            @pl.when(g * GROUP + b < n)
            def _(b=b):
                j = g * GROUP + b
                pltpu.make_async_copy(emb_hbm.at[isrc_v.at[delta + j]],
                                      bs_v.at[pl.ds(b * CHUNK, CHUNK)],
                                      sem_g).wait()
                pltpu.make_async_copy(emb_hbm.at[idst_v.at[delta + j]],
                                      bd_v.at[pl.ds(b * CHUNK, CHUNK)],
                                      sem_g).wait()

        @pl.when(g * GROUP + GROUP <= n)
        def _():
            base = (start + g * GROUP) * CHUNK
            pltpu.sync_copy(bs_v, gs_hbm.at[pl.ds(base, GROUP * CHUNK)])
            pltpu.sync_copy(bd_v, gd_hbm.at[pl.ds(base, GROUP * CHUNK)])

        @pl.when(jnp.logical_and(g * GROUP < n, g * GROUP + GROUP > n))
        def _():
            for b in range(GROUP):
                @pl.when(g * GROUP + b < n)
                def _(b=b):
                    j = g * GROUP + b
                    base = (start + j) * CHUNK
                    pltpu.sync_copy(bs_v.at[pl.ds(b * CHUNK, CHUNK)],
                                    gs_hbm.at[pl.ds(base, CHUNK)])
                    pltpu.sync_copy(bd_v.at[pl.ds(b * CHUNK, CHUNK)],
                                    gd_hbm.at[pl.ds(base, CHUNK)])


def _gath_call(emb, eidx3):
    k = pl.kernel(
        _gath_body,
        out_type=(jax.ShapeDtypeStruct((E, D_H), _f32),
                  jax.ShapeDtypeStruct((E, D_H), _f32)),
        mesh=_mesh,
        scratch_types=[
            pltpu.VMEM((MAXROWS, CHUNK), jnp.int32),
            pltpu.VMEM((MAXROWS, CHUNK), jnp.int32),
            pltpu.VMEM((GROUP * CHUNK, D_H), _f32),
            pltpu.VMEM((GROUP * CHUNK, D_H), _f32),
            pltpu.SemaphoreType.DMA,
        ],
        compiler_params=_SC_PARAMS,
    )
    return k(emb, eidx3)


_BR = 256
_NB = G4 // _BR


def _matvec(w, v):
    return lax.dot_general(w, v, (((1,), (0,)), ((), ())),
                           preferred_element_type=_f32)


def _lstm_body(wih_ref, whh_ref, cur0_ref, bih_ref, bhh_ref, hout_ref,
               gates_ref, wsum_ref, h_ref, c_ref):
    t = pl.program_id(0)
    nb = pl.program_id(1)
    rows = pl.ds(nb * _BR, _BR)

    @pl.when(jnp.logical_and(t == 0, nb == 0))
    def _():
        h_ref[...] = jnp.zeros_like(h_ref)
        c_ref[...] = jnp.zeros_like(c_ref)

    @pl.when(t == 0)
    def _():
        wih = wih_ref[...]
        whh = whh_ref[...]
        wsum_ref[rows, :] = (wih + whh).astype(_bf16)
        gates_ref[rows, :] = _matvec(wih, cur0_ref[...])

    @pl.when(t > 0)
    def _():
        w = wsum_ref[rows, :]
        hv = h_ref[...].astype(_bf16)
        gates_ref[rows, :] = _matvec(w, hv)

    @pl.when(nb == _NB - 1)
    def _():
        gf = gates_ref[...] + bih_ref[...] + bhh_ref[...]
        i = jax.nn.sigmoid(gf[0:FLAT])
        f = jax.nn.sigmoid(gf[FLAT:2 * FLAT])
        gg = jnp.tanh(gf[2 * FLAT:3 * FLAT])
        o = jax.nn.sigmoid(gf[3 * FLAT:4 * FLAT])
        c_new = f * c_ref[...] + i * gg
        h_new = o * jnp.tanh(c_new)
        c_ref[...] = c_new
        h_ref[...] = h_new

        @pl.when(t == T - 1)
        def _():
            hout_ref[...] = h_new


def _lstm_call(W_ih, W_hh, cur0, b_ih, b_hh):
    wmap = lambda t, nb: (jnp.where(t == 0, nb, _NB - 1), 0)
    return pl.pallas_call(
        _lstm_body,
        grid=(T, _NB),
        in_specs=[
            pl.BlockSpec((_BR, FLAT), wmap),
            pl.BlockSpec((_BR, FLAT), wmap),
            pl.BlockSpec((FLAT, 1), lambda t, nb: (0, 0)),
            pl.BlockSpec((G4, 1), lambda t, nb: (0, 0)),
            pl.BlockSpec((G4, 1), lambda t, nb: (0, 0)),
        ],
        out_specs=pl.BlockSpec((FLAT, 1), lambda t, nb: (0, 0)),
        out_shape=jax.ShapeDtypeStruct((FLAT, 1), _f32),
        scratch_shapes=[
            pltpu.VMEM((G4, 1), _f32),
            pltpu.VMEM((G4, FLAT), _bf16),
            pltpu.VMEM((FLAT, 1), _f32),
            pltpu.VMEM((FLAT, 1), _f32),
        ],
        compiler_params=pltpu.CompilerParams(
            dimension_semantics=("arbitrary", "arbitrary")),
    )(W_ih, W_hh, cur0, b_ih, b_hh)


_BN = 2048


def _ydis_body(x_ref, w_ref, da_ref, db_ref, y_ref, dis_ref):
    deg = da_ref[...][:, 0:1] + db_ref[...][:, 0:1] + 1.0
    dis = lax.rsqrt(deg)
    xw = jnp.dot(x_ref[...], w_ref[...], preferred_element_type=_f32)
    dis_ref[...] = dis
    y_ref[...] = dis * xw


def _ydis_call(x2p, w_fin, degA, degB):
    return pl.pallas_call(
        _ydis_body,
        grid=(NPAD // _BN,),
        in_specs=[
            pl.BlockSpec((_BN, D_IN), lambda i: (i, 0)),
            pl.BlockSpec((D_IN, D_H), lambda i: (0, 0)),
            pl.BlockSpec((_BN, D_H), lambda i: (i, 0)),
            pl.BlockSpec((_BN, D_H), lambda i: (i, 0)),
        ],
        out_specs=[
            pl.BlockSpec((_BN, D_H), lambda i: (i, 0)),
            pl.BlockSpec((_BN, 1), lambda i: (i, 0)),
        ],
        out_shape=[
            jax.ShapeDtypeStruct((NPAD, D_H), _f32),
            jax.ShapeDtypeStruct((NPAD, 1), _f32),
        ],
    )(x2p, w_fin, degA, degB)


def _emb_body(sa_ref, sb_ref, y_ref, dis_ref, emb_ref):
    tot = sa_ref[...] + sb_ref[...] + y_ref[...]
    emb_ref[...] = jnp.maximum(dis_ref[...] * tot, 0.0)


def _emb_call(sA, sB, y, dis):
    return pl.pallas_call(
        _emb_body,
        grid=(NPAD // _BN,),
        in_specs=[
            pl.BlockSpec((_BN, D_H), lambda i: (i, 0)),
            pl.BlockSpec((_BN, D_H), lambda i: (i, 0)),
            pl.BlockSpec((_BN, D_H), lambda i: (i, 0)),
            pl.BlockSpec((_BN, 1), lambda i: (i, 0)),
        ],
        out_specs=pl.BlockSpec((_BN, D_H), lambda i: (i, 0)),
        out_shape=jax.ShapeDtypeStruct((NPAD, D_H), _f32),
    )(sA, sB, y, dis)


_BE = 8192


def _mlp_body(gs_ref, gd_ref, at_ref, w1_ref, b1_ref, w2_ref, b2_ref, out_ref):
    w1 = w1_ref[...]
    dot = lambda a, b: jnp.dot(a, b, preferred_element_type=_f32)
    hid = (dot(gs_ref[...], w1[0:D_H]) + dot(gd_ref[...], w1[D_H:2 * D_H])
           + dot(at_ref[...], w1[2 * D_H:3 * D_H]) + b1_ref[...])
    hid = jnp.maximum(hid, 0.0)
    out_ref[...] = dot(hid, w2_ref[...]) + b2_ref[...]


def _mlp_call(gs, gd, attr, w1, b1, w2, b2):
    return pl.pallas_call(
        _mlp_body,
        grid=(pl.cdiv(E, _BE),),
        in_specs=[
            pl.BlockSpec((_BE, D_H), lambda i: (i, 0)),
            pl.BlockSpec((_BE, D_H), lambda i: (i, 0)),
            pl.BlockSpec((_BE, D_H), lambda i: (i, 0)),
            pl.BlockSpec((3 * D_H, D_H), lambda i: (0, 0)),
            pl.BlockSpec((1, D_H), lambda i: (0, 0)),
            pl.BlockSpec((D_H, 1), lambda i: (0, 0)),
            pl.BlockSpec((1, 1), lambda i: (0, 0)),
        ],
        out_specs=pl.BlockSpec((_BE, 1), lambda i: (i, 0)),
        out_shape=jax.ShapeDtypeStruct((E, 1), _f32),
    )(gs, gd, attr, w1, b1, w2, b2)


def kernel(x, edge_index, edge_attr, initial_weights, W_ih, W_hh, b_ih, b_hh,
           mlp_W1, mlp_b1, mlp_W2, mlp_b2):
    eidx3 = edge_index.reshape(2, NCHUNKS, CHUNK)

    zeros_n16 = jnp.zeros((NPAD, D_H), _f32)
    ones_r = jnp.ones((CHUNK, D_H), _f32)

    deg2 = _deg_call(eidx3, zeros_n16, ones_r)
    h3 = _lstm_call(W_ih, W_hh, initial_weights.reshape(FLAT, 1),
                    b_ih.reshape(G4, 1), b_hh.reshape(G4, 1))
    w_fin = h3.reshape(D_IN, D_H)

    x2p = jnp.pad(x[T - 1], ((0, NPAD - N), (0, 0)))
    y, dis = _ydis_call(x2p, w_fin, deg2[0], deg2[1])

    s2 = _agg_call(y, eidx3, zeros_n16)
    emb = _emb_call(s2[0], s2[1], y, dis)
    gs, gd = _gath_call(emb, eidx3)

    logits = _mlp_call(gs, gd, edge_attr, mlp_W1,
                       mlp_b1.reshape(1, D_H), mlp_W2,
                       mlp_b2.reshape(1, 1))
    return logits.reshape(E)


# speedup vs baseline: 19.2081x; 1.8251x over previous
import jax
import jax.numpy as jnp
from jax import lax
from jax.experimental import pallas as pl
from jax.experimental.pallas import tpu as pltpu
from jax.experimental.pallas import tpu_sc as plsc

N = 10000
E = 320000
T = 3
D_IN = 128
D_H = 16
FLAT = D_IN * D_H
G4 = 4 * FLAT

NTILES = 32
CHUNK = 128
NCHUNKS = E // CHUNK
BASE_PT = NCHUNKS // NTILES
EXTRA = NCHUNKS - BASE_PT * NTILES
MAXROWS = BASE_PT + 1
GROUP = 16
NGROUP = (MAXROWS + GROUP - 1) // GROUP
NPAD = 10240
STRIPE = NPAD // 16

_f32 = jnp.float32
_bf16 = jnp.bfloat16
_mesh = plsc.VectorSubcoreMesh(core_axis_name="c", subcore_axis_name="s")
_SC_PARAMS = pltpu.CompilerParams(use_tc_tiling_on_sc=False)


def _tile_range(wid):
    start = BASE_PT * wid + jnp.minimum(wid, EXTRA)
    n = BASE_PT + jnp.where(wid < EXTRA, 1, 0)
    sstart = jnp.minimum(start, NCHUNKS - MAXROWS)
    return start, n, sstart, start - sstart


def _deg_body(eidx_hbm, zeros_hbm, ones_hbm, out_hbm, idx_v, ones_v, deg_sh,
              sem):
    c = lax.axis_index("c")
    s = lax.axis_index("s")
    wid = s * 2 + c
    start, n, sstart, delta = _tile_range(wid)
    pltpu.sync_copy(eidx_hbm.at[1, pl.ds(sstart, MAXROWS)], idx_v)
    pltpu.sync_copy(ones_hbm, ones_v)
    pltpu.sync_copy(zeros_hbm.at[pl.ds(s * STRIPE, STRIPE)],
                    deg_sh.at[pl.ds(s * STRIPE, STRIPE)])
    plsc.subcore_barrier()

    @pl.loop(0, NGROUP)
    def _(g):
        for b in range(GROUP):
            @pl.when(g * GROUP + b < n)
            def _(b=b):
                j = g * GROUP + b
                pltpu.async_copy(ones_v, deg_sh.at[idx_v.at[delta + j]],
                                 sem, add=True)
        for b in range(GROUP):
            @pl.when(g * GROUP + b < n)
            def _(b=b):
                j = g * GROUP + b
                pltpu.make_async_copy(ones_v, deg_sh.at[idx_v.at[delta + j]],
                                      sem).wait()

    plsc.subcore_barrier()
    pltpu.sync_copy(deg_sh.at[pl.ds(s * STRIPE, STRIPE)],
                    out_hbm.at[c, pl.ds(s * STRIPE, STRIPE)])


def _deg_call(eidx3, zeros_n16, ones_r):
    k = pl.kernel(
        _deg_body,
        out_type=jax.ShapeDtypeStruct((2, NPAD, D_H), _f32),
        mesh=_mesh,
        scratch_types=[
            pltpu.VMEM((MAXROWS, CHUNK), jnp.int32),
            pltpu.VMEM((CHUNK, D_H), _f32),
            pltpu.VMEM_SHARED((NPAD, D_H), _f32),
            pltpu.SemaphoreType.DMA,
        ],
        compiler_params=_SC_PARAMS,
    )
    return k(eidx3, zeros_n16, ones_r)


def _agg_body(y_hbm, eidx_hbm, zeros_hbm, out_hbm,
              isrc_v, idst_v, rows_v, acc_sh, sem_g, sem_s):
    c = lax.axis_index("c")
    s = lax.axis_index("s")
    wid = s * 2 + c
    start, n, sstart, delta = _tile_range(wid)
    pltpu.sync_copy(eidx_hbm.at[0, pl.ds(sstart, MAXROWS)], isrc_v)
    pltpu.sync_copy(eidx_hbm.at[1, pl.ds(sstart, MAXROWS)], idst_v)
    pltpu.sync_copy(zeros_hbm.at[pl.ds(s * STRIPE, STRIPE)],
                    acc_sh.at[pl.ds(s * STRIPE, STRIPE)])
    plsc.subcore_barrier()

    @pl.loop(0, NGROUP)
    def _(g):
        for b in range(GROUP):
            @pl.when(g * GROUP + b < n)
            def _(b=b):
                j = g * GROUP + b
                pltpu.async_copy(y_hbm.at[isrc_v.at[delta + j]],
                                 rows_v.at[pl.ds(b * CHUNK, CHUNK)], sem_g)
        for b in range(GROUP):
            @pl.when(g * GROUP + b < n)
            def _(b=b):
                j = g * GROUP + b
                pltpu.make_async_copy(y_hbm.at[isrc_v.at[delta + j]],
                                      rows_v.at[pl.ds(b * CHUNK, CHUNK)],
                                      sem_g).wait()
        for b in range(GROUP):
            @pl.when(g * GROUP + b < n)
            def _(b=b):
                j = g * GROUP + b
                pltpu.async_copy(rows_v.at[pl.ds(b * CHUNK, CHUNK)],
                                 acc_sh.at[idst_v.at[delta + j]], sem_s,
                                 add=True)
        for b in range(GROUP):
            @pl.when(g * GROUP + b < n)
            def _(b=b):
                j = g * GROUP + b
                pltpu.make_async_copy(rows_v.at[pl.ds(b * CHUNK, CHUNK)],
                                      acc_sh.at[idst_v.at[delta + j]],
                                      sem_s).wait()

    plsc.subcore_barrier()
    pltpu.sync_copy(acc_sh.at[pl.ds(s * STRIPE, STRIPE)],
                    out_hbm.at[c, pl.ds(s * STRIPE, STRIPE)])


def _agg_call(y, eidx3, zeros_n16):
    k = pl.kernel(
        _agg_body,
        out_type=jax.ShapeDtypeStruct((2, NPAD, D_H), _f32),
        mesh=_mesh,
        scratch_types=[
            pltpu.VMEM((MAXROWS, CHUNK), jnp.int32),
            pltpu.VMEM((MAXROWS, CHUNK), jnp.int32),
            pltpu.VMEM((GROUP * CHUNK, D_H), _f32),
            pltpu.VMEM_SHARED((NPAD, D_H), _f32),
            pltpu.SemaphoreType.DMA,
            pltpu.SemaphoreType.DMA,
        ],
        compiler_params=_SC_PARAMS,
    )
    return k(y, eidx3, zeros_n16)


def _gath_body(emb_hbm, eidx_hbm, gs_hbm, gd_hbm,
               isrc_v, idst_v, bs_v, bd_v, sem_g):
    c = lax.axis_index("c")
    s = lax.axis_index("s")
    wid = s * 2 + c
    start, n, sstart, delta = _tile_range(wid)
    pltpu.sync_copy(eidx_hbm.at[0, pl.ds(sstart, MAXROWS)], isrc_v)
    pltpu.sync_copy(eidx_hbm.at[1, pl.ds(sstart, MAXROWS)], idst_v)

    @pl.loop(0, NGROUP)
    def _(g):
        for b in range(GROUP):
            @pl.when(g * GROUP + b < n)
            def _(b=b):
                j = g * GROUP + b
                pltpu.async_copy(emb_hbm.at[isrc_v.at[delta + j]],
                                 bs_v.at[pl.ds(b * CHUNK, CHUNK)], sem_g)
                pltpu.async_copy(emb_hbm.at[idst_v.at[delta + j]],
                                 bd_v.at[pl.ds(b * CHUNK, CHUNK)], sem_g)
        for b in range(GROUP):
            @pl.when(g * GROUP + b < n)
            def _(b=b):
                j = g * GROUP + b
                pltpu.make_async_copy(emb_hbm.at[isrc_v.at[delta + j]],
                                      bs_v.at[pl.ds(b * CHUNK, CHUNK)],
                                      sem_g).wait()
                pltpu.make_async_copy(emb_hbm.at[idst_v.at[delta + j]],
                                      bd_v.at[pl.ds(b * CHUNK, CHUNK)],
                                      sem_g).wait()

        @pl.when(g * GROUP + GROUP <= n)
        def _():
            base = (start + g * GROUP) * CHUNK
            pltpu.sync_copy(bs_v, gs_hbm.at[pl.ds(base, GROUP * CHUNK)])
            pltpu.sync_copy(bd_v, gd_hbm.at[pl.ds(base, GROUP * CHUNK)])

        @pl.when(jnp.logical_and(g * GROUP < n, g * GROUP + GROUP > n))
        def _():
            for b in range(GROUP):
                @pl.when(g * GROUP + b < n)
                def _(b=b):
                    j = g * GROUP + b
                    base = (start + j) * CHUNK
                    pltpu.sync_copy(bs_v.at[pl.ds(b * CHUNK, CHUNK)],
                                    gs_hbm.at[pl.ds(base, CHUNK)])
                    pltpu.sync_copy(bd_v.at[pl.ds(b * CHUNK, CHUNK)],
                                    gd_hbm.at[pl.ds(base, CHUNK)])


def _gath_call(emb, eidx3):
    k = pl.kernel(
        _gath_body,
        out_type=(jax.ShapeDtypeStruct((E, D_H), _f32),
                  jax.ShapeDtypeStruct((E, D_H), _f32)),
        mesh=_mesh,
        scratch_types=[
            pltpu.VMEM((MAXROWS, CHUNK), jnp.int32),
            pltpu.VMEM((MAXROWS, CHUNK), jnp.int32),
            pltpu.VMEM((GROUP * CHUNK, D_H), _f32),
            pltpu.VMEM((GROUP * CHUNK, D_H), _f32),
            pltpu.SemaphoreType.DMA,
        ],
        compiler_params=_SC_PARAMS,
    )
    return k(emb, eidx3)


_BR = 256
_NB = G4 // _BR


def _matvec(w, v):
    return lax.dot_general(w, v, (((1,), (0,)), ((), ())),
                           preferred_element_type=_f32)


def _lstm_body(wih_ref, whh_ref, cur0_ref, bih_ref, bhh_ref, hout_ref,
               gates_ref, wsum_ref, h_ref, c_ref):
    t = pl.program_id(0)
    nb = pl.program_id(1)
    rows = pl.ds(nb * _BR, _BR)

    @pl.when(jnp.logical_and(t == 0, nb == 0))
    def _():
        h_ref[...] = jnp.zeros_like(h_ref)
        c_ref[...] = jnp.zeros_like(c_ref)

    @pl.when(t == 0)
    def _():
        wih = wih_ref[...]
        whh = whh_ref[...]
        wsum_ref[rows, :] = (wih + whh).astype(_bf16)
        gates_ref[rows, :] = _matvec(wih, cur0_ref[...])

    @pl.when(t > 0)
    def _():
        w = wsum_ref[rows, :]
        hv = h_ref[...].astype(_bf16)
        gates_ref[rows, :] = _matvec(w, hv)

    @pl.when(nb == _NB - 1)
    def _():
        gf = gates_ref[...] + bih_ref[...] + bhh_ref[...]
        i = jax.nn.sigmoid(gf[0:FLAT])
        f = jax.nn.sigmoid(gf[FLAT:2 * FLAT])
        gg = jnp.tanh(gf[2 * FLAT:3 * FLAT])
        o = jax.nn.sigmoid(gf[3 * FLAT:4 * FLAT])
        c_new = f * c_ref[...] + i * gg
        h_new = o * jnp.tanh(c_new)
        c_ref[...] = c_new
        h_ref[...] = h_new

        @pl.when(t == T - 1)
        def _():
            hout_ref[...] = h_new


def _lstm_call(W_ih, W_hh, cur0, b_ih, b_hh):
    wmap = lambda t, nb: (jnp.where(t == 0, nb, _NB - 1), 0)
    return pl.pallas_call(
        _lstm_body,
        grid=(T, _NB),
        in_specs=[
            pl.BlockSpec((_BR, FLAT), wmap),
            pl.BlockSpec((_BR, FLAT), wmap),
            pl.BlockSpec((FLAT, 1), lambda t, nb: (0, 0)),
            pl.BlockSpec((G4, 1), lambda t, nb: (0, 0)),
            pl.BlockSpec((G4, 1), lambda t, nb: (0, 0)),
        ],
        out_specs=pl.BlockSpec((FLAT, 1), lambda t, nb: (0, 0)),
        out_shape=jax.ShapeDtypeStruct((FLAT, 1), _f32),
        scratch_shapes=[
            pltpu.VMEM((G4, 1), _f32),
            pltpu.VMEM((G4, FLAT), _bf16),
            pltpu.VMEM((FLAT, 1), _f32),
            pltpu.VMEM((FLAT, 1), _f32),
        ],
        compiler_params=pltpu.CompilerParams(
            dimension_semantics=("arbitrary", "arbitrary")),
    )(W_ih, W_hh, cur0, b_ih, b_hh)


_BN = 2048


def _ydis_body(x_ref, w_ref, da_ref, db_ref, y_ref, dis_ref):
    deg = da_ref[...][:, 0:1] + db_ref[...][:, 0:1] + 1.0
    dis = lax.rsqrt(deg)
    xw = jnp.dot(x_ref[...], w_ref[...], preferred_element_type=_f32)
    dis_ref[...] = dis
    y_ref[...] = dis * xw


def _ydis_call(x2p, w_fin, degA, degB):
    return pl.pallas_call(
        _ydis_body,
        grid=(NPAD // _BN,),
        in_specs=[
            pl.BlockSpec((_BN, D_IN), lambda i: (i, 0)),
            pl.BlockSpec((D_IN, D_H), lambda i: (0, 0)),
            pl.BlockSpec((_BN, D_H), lambda i: (i, 0)),
            pl.BlockSpec((_BN, D_H), lambda i: (i, 0)),
        ],
        out_specs=[
            pl.BlockSpec((_BN, D_H), lambda i: (i, 0)),
            pl.BlockSpec((_BN, 1), lambda i: (i, 0)),
        ],
        out_shape=[
            jax.ShapeDtypeStruct((NPAD, D_H), _f32),
            jax.ShapeDtypeStruct((NPAD, 1), _f32),
        ],
    )(x2p, w_fin, degA, degB)


def _emb_body(sa_ref, sb_ref, y_ref, dis_ref, emb_ref):
    tot = sa_ref[...] + sb_ref[...] + y_ref[...]
    emb_ref[...] = jnp.maximum(dis_ref[...] * tot, 0.0)


def _emb_call(sA, sB, y, dis):
    return pl.pallas_call(
        _emb_body,
        grid=(NPAD // _BN,),
        in_specs=[
            pl.BlockSpec((_BN, D_H), lambda i: (i, 0)),
            pl.BlockSpec((_BN, D_H), lambda i: (i, 0)),
            pl.BlockSpec((_BN, D_H), lambda i: (i, 0)),
            pl.BlockSpec((_BN, 1), lambda i: (i, 0)),
        ],
        out_specs=pl.BlockSpec((_BN, D_H), lambda i: (i, 0)),
        out_shape=jax.ShapeDtypeStruct((NPAD, D_H), _f32),
    )(sA, sB, y, dis)


_BE = 8192
_BR8 = _BE // 8
_NLANE = 8 * D_H


def _mlp_body(gs_ref, gd_ref, at_ref, bd1_ref, b1_ref, bd2_ref, b2_ref,
              out_ref):
    dot = lambda a, b: jnp.dot(a, b, preferred_element_type=_f32)
    bd1 = bd1_ref[...]
    hid = (dot(gs_ref[...], bd1[0:_NLANE]) + dot(gd_ref[...], bd1[_NLANE:2 * _NLANE])
           + dot(at_ref[...], bd1[2 * _NLANE:3 * _NLANE]) + b1_ref[...])
    hid = jnp.maximum(hid, 0.0)
    out_ref[...] = dot(hid, bd2_ref[...]) + b2_ref[...]


def _mlp_call(gs8, gd8, at8, bd1, b1p, bd2, b2):
    return pl.pallas_call(
        _mlp_body,
        grid=(pl.cdiv(E, _BE),),
        in_specs=[
            pl.BlockSpec((_BR8, _NLANE), lambda i: (i, 0)),
            pl.BlockSpec((_BR8, _NLANE), lambda i: (i, 0)),
            pl.BlockSpec((_BR8, _NLANE), lambda i: (i, 0)),
            pl.BlockSpec((3 * _NLANE, _NLANE), lambda i: (0, 0)),
            pl.BlockSpec((1, _NLANE), lambda i: (0, 0)),
            pl.BlockSpec((_NLANE, 8), lambda i: (0, 0)),
            pl.BlockSpec((1, 1), lambda i: (0, 0)),
        ],
        out_specs=pl.BlockSpec((_BR8, 8), lambda i: (i, 0)),
        out_shape=jax.ShapeDtypeStruct((E // 8, 8), _f32),
    )(gs8, gd8, at8, bd1, b1p, bd2, b2)


def kernel(x, edge_index, edge_attr, initial_weights, W_ih, W_hh, b_ih, b_hh,
           mlp_W1, mlp_b1, mlp_W2, mlp_b2):
    eidx3 = edge_index.reshape(2, NCHUNKS, CHUNK)

    zeros_n16 = jnp.zeros((NPAD, D_H), _f32)
    ones_r = jnp.ones((CHUNK, D_H), _f32)

    deg2 = _deg_call(eidx3, zeros_n16, ones_r)
    h3 = _lstm_call(W_ih, W_hh, initial_weights.reshape(FLAT, 1),
                    b_ih.reshape(G4, 1), b_hh.reshape(G4, 1))
    w_fin = h3.reshape(D_IN, D_H)

    x2p = jnp.pad(x[T - 1], ((0, NPAD - N), (0, 0)))
    y, dis = _ydis_call(x2p, w_fin, deg2[0], deg2[1])

    s2 = _agg_call(y, eidx3, zeros_n16)
    emb = _emb_call(s2[0], s2[1], y, dis)
    gs, gd = _gath_call(emb, eidx3)

    eye8 = jnp.eye(8, dtype=_f32)
    bd1 = jnp.concatenate([jnp.kron(eye8, mlp_W1[k * D_H:(k + 1) * D_H])
                           for k in range(3)], axis=0)
    bd2 = jnp.kron(eye8, mlp_W2)
    b1p = jnp.tile(mlp_b1, 8).reshape(1, 8 * D_H)
    logits = _mlp_call(gs.reshape(E // 8, 8 * D_H),
                       gd.reshape(E // 8, 8 * D_H),
                       edge_attr.reshape(E // 8, 8 * D_H),
                       bd1, b1p, bd2, mlp_b2.reshape(1, 1))
    return logits.reshape(E)


# speedup vs baseline: 20.3872x; 1.0614x over previous
import jax
import jax.numpy as jnp
from jax import lax
from jax.experimental import pallas as pl
from jax.experimental.pallas import tpu as pltpu
from jax.experimental.pallas import tpu_sc as plsc

N = 10000
E = 320000
T = 3
D_IN = 128
D_H = 16
FLAT = D_IN * D_H
G4 = 4 * FLAT

NTILES = 32
CHUNK = 128
NCHUNKS = E // CHUNK
BASE_PT = NCHUNKS // NTILES
EXTRA = NCHUNKS - BASE_PT * NTILES
MAXROWS = BASE_PT + 1
GROUP = 16
NGROUP = (MAXROWS + GROUP - 1) // GROUP
NPAD = 10240
STRIPE = NPAD // 16

_f32 = jnp.float32
_bf16 = jnp.bfloat16
_mesh = plsc.VectorSubcoreMesh(core_axis_name="c", subcore_axis_name="s")
_SC_PARAMS = pltpu.CompilerParams(use_tc_tiling_on_sc=False)


def _tile_range(wid):
    start = BASE_PT * wid + jnp.minimum(wid, EXTRA)
    n = BASE_PT + jnp.where(wid < EXTRA, 1, 0)
    sstart = jnp.minimum(start, NCHUNKS - MAXROWS)
    return start, n, sstart, start - sstart


def _deg_body(eidx_hbm, zeros_hbm, ones_hbm, out_hbm, idx_v, ones_v, deg_sh,
              sem):
    c = lax.axis_index("c")
    s = lax.axis_index("s")
    wid = s * 2 + c
    start, n, sstart, delta = _tile_range(wid)
    pltpu.sync_copy(eidx_hbm.at[1, pl.ds(sstart, MAXROWS)], idx_v)
    pltpu.sync_copy(ones_hbm, ones_v)
    pltpu.sync_copy(zeros_hbm.at[pl.ds(s * STRIPE, STRIPE)],
                    deg_sh.at[pl.ds(s * STRIPE, STRIPE)])
    plsc.subcore_barrier()

    @pl.loop(0, NGROUP)
    def _(g):
        for b in range(GROUP):
            @pl.when(g * GROUP + b < n)
            def _(b=b):
                j = g * GROUP + b
                pltpu.async_copy(ones_v, deg_sh.at[idx_v.at[delta + j]],
                                 sem, add=True)
        for b in range(GROUP):
            @pl.when(g * GROUP + b < n)
            def _(b=b):
                j = g * GROUP + b
                pltpu.make_async_copy(ones_v, deg_sh.at[idx_v.at[delta + j]],
                                      sem).wait()

    plsc.subcore_barrier()
    pltpu.sync_copy(deg_sh.at[pl.ds(s * STRIPE, STRIPE)],
                    out_hbm.at[c, pl.ds(s * STRIPE, STRIPE)])


def _deg_call(eidx3, zeros_n16, ones_r):
    k = pl.kernel(
        _deg_body,
        out_type=jax.ShapeDtypeStruct((2, NPAD, D_H), _f32),
        mesh=_mesh,
        scratch_types=[
            pltpu.VMEM((MAXROWS, CHUNK), jnp.int32),
            pltpu.VMEM((CHUNK, D_H), _f32),
            pltpu.VMEM_SHARED((NPAD, D_H), _f32),
            pltpu.SemaphoreType.DMA,
        ],
        compiler_params=_SC_PARAMS,
    )
    return k(eidx3, zeros_n16, ones_r)


def _agg_body(y_hbm, eidx_hbm, zeros_hbm, out_hbm,
              isrc_v, idst_v, rows_v, acc_sh, y_sh, sem_g, sem_s):
    c = lax.axis_index("c")
    s = lax.axis_index("s")
    wid = s * 2 + c
    start, n, sstart, delta = _tile_range(wid)
    pltpu.sync_copy(eidx_hbm.at[0, pl.ds(sstart, MAXROWS)], isrc_v)
    pltpu.sync_copy(eidx_hbm.at[1, pl.ds(sstart, MAXROWS)], idst_v)
    pltpu.sync_copy(zeros_hbm.at[pl.ds(s * STRIPE, STRIPE)],
                    acc_sh.at[pl.ds(s * STRIPE, STRIPE)])
    pltpu.sync_copy(y_hbm.at[pl.ds(s * STRIPE, STRIPE)],
                    y_sh.at[pl.ds(s * STRIPE, STRIPE)])
    plsc.subcore_barrier()

    @pl.loop(0, NGROUP)
    def _(g):
        for b in range(GROUP):
            @pl.when(g * GROUP + b < n)
            def _(b=b):
                j = g * GROUP + b
                pltpu.async_copy(y_sh.at[isrc_v.at[delta + j]],
                                 rows_v.at[pl.ds(b * CHUNK, CHUNK)], sem_g)
        for b in range(GROUP):
            @pl.when(g * GROUP + b < n)
            def _(b=b):
                j = g * GROUP + b
                pltpu.make_async_copy(y_sh.at[isrc_v.at[delta + j]],
                                      rows_v.at[pl.ds(b * CHUNK, CHUNK)],
                                      sem_g).wait()
        for b in range(GROUP):
            @pl.when(g * GROUP + b < n)
            def _(b=b):
                j = g * GROUP + b
                pltpu.async_copy(rows_v.at[pl.ds(b * CHUNK, CHUNK)],
                                 acc_sh.at[idst_v.at[delta + j]], sem_s,
                                 add=True)
        for b in range(GROUP):
            @pl.when(g * GROUP + b < n)
            def _(b=b):
                j = g * GROUP + b
                pltpu.make_async_copy(rows_v.at[pl.ds(b * CHUNK, CHUNK)],
                                      acc_sh.at[idst_v.at[delta + j]],
                                      sem_s).wait()

    plsc.subcore_barrier()
    pltpu.sync_copy(acc_sh.at[pl.ds(s * STRIPE, STRIPE)],
                    out_hbm.at[c, pl.ds(s * STRIPE, STRIPE)])


def _agg_call(y, eidx3, zeros_n16):
    k = pl.kernel(
        _agg_body,
        out_type=jax.ShapeDtypeStruct((2, NPAD, D_H), _f32),
        mesh=_mesh,
        scratch_types=[
            pltpu.VMEM((MAXROWS, CHUNK), jnp.int32),
            pltpu.VMEM((MAXROWS, CHUNK), jnp.int32),
            pltpu.VMEM((GROUP * CHUNK, D_H), _f32),
            pltpu.VMEM_SHARED((NPAD, D_H), _f32),
            pltpu.VMEM_SHARED((NPAD, D_H), _f32),
            pltpu.SemaphoreType.DMA,
            pltpu.SemaphoreType.DMA,
        ],
        compiler_params=_SC_PARAMS,
    )
    return k(y, eidx3, zeros_n16)


def _gath_body(emb_hbm, eidx_hbm, gs_hbm, gd_hbm,
               isrc_v, idst_v, bs_v, bd_v, emb_sh, sem_g):
    c = lax.axis_index("c")
    s = lax.axis_index("s")
    wid = s * 2 + c
    start, n, sstart, delta = _tile_range(wid)
    pltpu.sync_copy(eidx_hbm.at[0, pl.ds(sstart, MAXROWS)], isrc_v)
    pltpu.sync_copy(eidx_hbm.at[1, pl.ds(sstart, MAXROWS)], idst_v)
    pltpu.sync_copy(emb_hbm.at[pl.ds(s * STRIPE, STRIPE)],
                    emb_sh.at[pl.ds(s * STRIPE, STRIPE)])
    plsc.subcore_barrier()

    @pl.loop(0, NGROUP)
    def _(g):
        for b in range(GROUP):
            @pl.when(g * GROUP + b < n)
            def _(b=b):
                j = g * GROUP + b
                pltpu.async_copy(emb_sh.at[isrc_v.at[delta + j]],
                                 bs_v.at[pl.ds(b * CHUNK, CHUNK)], sem_g)
                pltpu.async_copy(emb_sh.at[idst_v.at[delta + j]],
                                 bd_v.at[pl.ds(b * CHUNK, CHUNK)], sem_g)
        for b in range(GROUP):
            @pl.when(g * GROUP + b < n)
            def _(b=b):
                j = g * GROUP + b
                pltpu.make_async_copy(emb_sh.at[isrc_v.at[delta + j]],
                                      bs_v.at[pl.ds(b * CHUNK, CHUNK)],
                                      sem_g).wait()
                pltpu.make_async_copy(emb_sh.at[idst_v.at[delta + j]],
                                      bd_v.at[pl.ds(b * CHUNK, CHUNK)],
                                      sem_g).wait()

        @pl.when(g * GROUP + GROUP <= n)
        def _():
            base = (start + g * GROUP) * CHUNK
            pltpu.sync_copy(bs_v, gs_hbm.at[pl.ds(base, GROUP * CHUNK)])
            pltpu.sync_copy(bd_v, gd_hbm.at[pl.ds(base, GROUP * CHUNK)])

        @pl.when(jnp.logical_and(g * GROUP < n, g * GROUP + GROUP > n))
        def _():
            for b in range(GROUP):
                @pl.when(g * GROUP + b < n)
                def _(b=b):
                    j = g * GROUP + b
                    base = (start + j) * CHUNK
                    pltpu.sync_copy(bs_v.at[pl.ds(b * CHUNK, CHUNK)],
                                    gs_hbm.at[pl.ds(base, CHUNK)])
                    pltpu.sync_copy(bd_v.at[pl.ds(b * CHUNK, CHUNK)],
                                    gd_hbm.at[pl.ds(base, CHUNK)])


def _gath_call(emb, eidx3):
    k = pl.kernel(
        _gath_body,
        out_type=(jax.ShapeDtypeStruct((E, D_H), _f32),
                  jax.ShapeDtypeStruct((E, D_H), _f32)),
        mesh=_mesh,
        scratch_types=[
            pltpu.VMEM((MAXROWS, CHUNK), jnp.int32),
            pltpu.VMEM((MAXROWS, CHUNK), jnp.int32),
            pltpu.VMEM((GROUP * CHUNK, D_H), _f32),
            pltpu.VMEM((GROUP * CHUNK, D_H), _f32),
            pltpu.VMEM_SHARED((NPAD, D_H), _f32),
            pltpu.SemaphoreType.DMA,
        ],
        compiler_params=_SC_PARAMS,
    )
    return k(emb, eidx3)


_BR = 256
_NB = G4 // _BR


def _matvec(w, v):
    return lax.dot_general(w, v, (((1,), (0,)), ((), ())),
                           preferred_element_type=_f32)


def _lstm_body(wih_ref, whh_ref, cur0_ref, bih_ref, bhh_ref, hout_ref,
               gates_ref, wsum_ref, h_ref, c_ref):
    t = pl.program_id(0)
    nb = pl.program_id(1)
    rows = pl.ds(nb * _BR, _BR)

    @pl.when(jnp.logical_and(t == 0, nb == 0))
    def _():
        h_ref[...] = jnp.zeros_like(h_ref)
        c_ref[...] = jnp.zeros_like(c_ref)

    @pl.when(t == 0)
    def _():
        wih = wih_ref[...]
        whh = whh_ref[...]
        wsum_ref[rows, :] = (wih + whh).astype(_bf16)
        gates_ref[rows, :] = _matvec(wih.astype(_bf16),
                                     cur0_ref[...].astype(_bf16))

    @pl.when(t > 0)
    def _():
        w = wsum_ref[rows, :]
        hv = h_ref[...].astype(_bf16)
        gates_ref[rows, :] = _matvec(w, hv)

    @pl.when(nb == _NB - 1)
    def _():
        gf = gates_ref[...] + bih_ref[...] + bhh_ref[...]
        i = jax.nn.sigmoid(gf[0:FLAT])
        f = jax.nn.sigmoid(gf[FLAT:2 * FLAT])
        gg = jnp.tanh(gf[2 * FLAT:3 * FLAT])
        o = jax.nn.sigmoid(gf[3 * FLAT:4 * FLAT])
        c_new = f * c_ref[...] + i * gg
        h_new = o * jnp.tanh(c_new)
        c_ref[...] = c_new
        h_ref[...] = h_new

        @pl.when(t == T - 1)
        def _():
            hout_ref[...] = h_new


def _lstm_call(W_ih, W_hh, cur0, b_ih, b_hh):
    wmap = lambda t, nb: (jnp.where(t == 0, nb, _NB - 1), 0)
    return pl.pallas_call(
        _lstm_body,
        grid=(T, _NB),
        in_specs=[
            pl.BlockSpec((_BR, FLAT), wmap),
            pl.BlockSpec((_BR, FLAT), wmap),
            pl.BlockSpec((FLAT, 1), lambda t, nb: (0, 0)),
            pl.BlockSpec((G4, 1), lambda t, nb: (0, 0)),
            pl.BlockSpec((G4, 1), lambda t, nb: (0, 0)),
        ],
        out_specs=pl.BlockSpec((FLAT, 1), lambda t, nb: (0, 0)),
        out_shape=jax.ShapeDtypeStruct((FLAT, 1), _f32),
        scratch_shapes=[
            pltpu.VMEM((G4, 1), _f32),
            pltpu.VMEM((G4, FLAT), _bf16),
            pltpu.VMEM((FLAT, 1), _f32),
            pltpu.VMEM((FLAT, 1), _f32),
        ],
        compiler_params=pltpu.CompilerParams(
            dimension_semantics=("arbitrary", "arbitrary")),
    )(W_ih, W_hh, cur0, b_ih, b_hh)


_BN = 2048


def _ydis_body(x_ref, w_ref, da_ref, db_ref, y_ref, dis_ref):
    deg = da_ref[...][:, 0:1] + db_ref[...][:, 0:1] + 1.0
    dis = lax.rsqrt(deg)
    xw = jnp.dot(x_ref[...], w_ref[...], preferred_element_type=_f32)
    dis_ref[...] = dis
    y_ref[...] = dis * xw


def _ydis_call(x2p, w_fin, degA, degB):
    return pl.pallas_call(
        _ydis_body,
        grid=(NPAD // _BN,),
        in_specs=[
            pl.BlockSpec((_BN, D_IN), lambda i: (i, 0)),
            pl.BlockSpec((D_IN, D_H), lambda i: (0, 0)),
            pl.BlockSpec((_BN, D_H), lambda i: (i, 0)),
            pl.BlockSpec((_BN, D_H), lambda i: (i, 0)),
        ],
        out_specs=[
            pl.BlockSpec((_BN, D_H), lambda i: (i, 0)),
            pl.BlockSpec((_BN, 1), lambda i: (i, 0)),
        ],
        out_shape=[
            jax.ShapeDtypeStruct((NPAD, D_H), _f32),
            jax.ShapeDtypeStruct((NPAD, 1), _f32),
        ],
    )(x2p, w_fin, degA, degB)


def _emb_body(sa_ref, sb_ref, y_ref, dis_ref, emb_ref):
    tot = sa_ref[...] + sb_ref[...] + y_ref[...]
    emb_ref[...] = jnp.maximum(dis_ref[...] * tot, 0.0)


def _emb_call(sA, sB, y, dis):
    return pl.pallas_call(
        _emb_body,
        grid=(NPAD // _BN,),
        in_specs=[
            pl.BlockSpec((_BN, D_H), lambda i: (i, 0)),
            pl.BlockSpec((_BN, D_H), lambda i: (i, 0)),
            pl.BlockSpec((_BN, D_H), lambda i: (i, 0)),
            pl.BlockSpec((_BN, 1), lambda i: (i, 0)),
        ],
        out_specs=pl.BlockSpec((_BN, D_H), lambda i: (i, 0)),
        out_shape=jax.ShapeDtypeStruct((NPAD, D_H), _f32),
    )(sA, sB, y, dis)


_BE = 8192
_BR8 = _BE // 8
_NLANE = 8 * D_H


def _mlp_body(gs_ref, gd_ref, at_ref, bd1_ref, b1_ref, bd2_ref, b2_ref,
              out_ref):
    dot = lambda a, b: jnp.dot(a, b, preferred_element_type=_f32)
    bd1 = bd1_ref[...]
    hid = (dot(gs_ref[...], bd1[0:_NLANE]) + dot(gd_ref[...], bd1[_NLANE:2 * _NLANE])
           + dot(at_ref[...], bd1[2 * _NLANE:3 * _NLANE]) + b1_ref[...])
    hid = jnp.maximum(hid, 0.0)
    out_ref[...] = dot(hid, bd2_ref[...]) + b2_ref[...]


def _mlp_call(gs8, gd8, at8, bd1, b1p, bd2, b2):
    return pl.pallas_call(
        _mlp_body,
        grid=(pl.cdiv(E, _BE),),
        in_specs=[
            pl.BlockSpec((_BR8, _NLANE), lambda i: (i, 0)),
            pl.BlockSpec((_BR8, _NLANE), lambda i: (i, 0)),
            pl.BlockSpec((_BR8, _NLANE), lambda i: (i, 0)),
            pl.BlockSpec((3 * _NLANE, _NLANE), lambda i: (0, 0)),
            pl.BlockSpec((1, _NLANE), lambda i: (0, 0)),
            pl.BlockSpec((_NLANE, 8), lambda i: (0, 0)),
            pl.BlockSpec((1, 1), lambda i: (0, 0)),
        ],
        out_specs=pl.BlockSpec((_BR8, 8), lambda i: (i, 0)),
        out_shape=jax.ShapeDtypeStruct((E // 8, 8), _f32),
    )(gs8, gd8, at8, bd1, b1p, bd2, b2)


def kernel(x, edge_index, edge_attr, initial_weights, W_ih, W_hh, b_ih, b_hh,
           mlp_W1, mlp_b1, mlp_W2, mlp_b2):
    eidx3 = edge_index.reshape(2, NCHUNKS, CHUNK)

    zeros_n16 = jnp.zeros((NPAD, D_H), _f32)
    ones_r = jnp.ones((CHUNK, D_H), _f32)

    deg2 = _deg_call(eidx3, zeros_n16, ones_r)
    h3 = _lstm_call(W_ih, W_hh, initial_weights.reshape(FLAT, 1),
                    b_ih.reshape(G4, 1), b_hh.reshape(G4, 1))
    w_fin = h3.reshape(D_IN, D_H)

    x2p = jnp.pad(x[T - 1], ((0, NPAD - N), (0, 0)))
    y, dis = _ydis_call(x2p, w_fin, deg2[0], deg2[1])

    s2 = _agg_call(y, eidx3, zeros_n16)
    emb = _emb_call(s2[0], s2[1], y, dis)
    gs, gd = _gath_call(emb, eidx3)

    eye8 = jnp.eye(8, dtype=_f32)
    bd1 = jnp.concatenate([jnp.kron(eye8, mlp_W1[k * D_H:(k + 1) * D_H])
                           for k in range(3)], axis=0)
    bd2 = jnp.kron(eye8, mlp_W2)
    b1p = jnp.tile(mlp_b1, 8).reshape(1, 8 * D_H)
    logits = _mlp_call(gs.reshape(E // 8, 8 * D_H),
                       gd.reshape(E // 8, 8 * D_H),
                       edge_attr.reshape(E // 8, 8 * D_H),
                       bd1, b1p, bd2, mlp_b2.reshape(1, 1))
    return logits.reshape(E)


# speedup vs baseline: 20.4502x; 1.0031x over previous
import jax
import jax.numpy as jnp
from jax import lax
from jax.experimental import pallas as pl
from jax.experimental.pallas import tpu as pltpu
from jax.experimental.pallas import tpu_sc as plsc

N = 10000
E = 320000
T = 3
D_IN = 128
D_H = 16
FLAT = D_IN * D_H
G4 = 4 * FLAT

NTILES = 32
CHUNK = 128
NCHUNKS = E // CHUNK
BASE_PT = NCHUNKS // NTILES
EXTRA = NCHUNKS - BASE_PT * NTILES
MAXROWS = BASE_PT + 1
GROUP = 16
NGROUP = (MAXROWS + GROUP - 1) // GROUP
NPAD = 10240
STRIPE = NPAD // 16

_f32 = jnp.float32
_bf16 = jnp.bfloat16
_mesh = plsc.VectorSubcoreMesh(core_axis_name="c", subcore_axis_name="s")
_SC_PARAMS = pltpu.CompilerParams(use_tc_tiling_on_sc=False)


def _tile_range(wid):
    start = BASE_PT * wid + jnp.minimum(wid, EXTRA)
    n = BASE_PT + jnp.where(wid < EXTRA, 1, 0)
    sstart = jnp.minimum(start, NCHUNKS - MAXROWS)
    return start, n, sstart, start - sstart


def _deg_body(eidx_hbm, zeros_hbm, ones_hbm, out_hbm, idx_v, ones_v, deg_sh,
              sem):
    c = lax.axis_index("c")
    s = lax.axis_index("s")
    wid = s * 2 + c
    start, n, sstart, delta = _tile_range(wid)
    pltpu.sync_copy(eidx_hbm.at[1, pl.ds(sstart, MAXROWS)], idx_v)
    pltpu.sync_copy(ones_hbm, ones_v)
    pltpu.sync_copy(zeros_hbm.at[pl.ds(s * STRIPE, STRIPE)],
                    deg_sh.at[pl.ds(s * STRIPE, STRIPE)])
    plsc.subcore_barrier()

    @pl.loop(0, NGROUP)
    def _(g):
        for b in range(GROUP):
            @pl.when(g * GROUP + b < n)
            def _(b=b):
                j = g * GROUP + b
                pltpu.async_copy(ones_v, deg_sh.at[idx_v.at[delta + j]],
                                 sem, add=True)
        for b in range(GROUP):
            @pl.when(g * GROUP + b < n)
            def _(b=b):
                j = g * GROUP + b
                pltpu.make_async_copy(ones_v, deg_sh.at[idx_v.at[delta + j]],
                                      sem).wait()

    plsc.subcore_barrier()
    pltpu.sync_copy(deg_sh.at[pl.ds(s * STRIPE, STRIPE)],
                    out_hbm.at[c, pl.ds(s * STRIPE, STRIPE)])


def _deg_call(eidx3, zeros_n16, ones_r):
    k = pl.kernel(
        _deg_body,
        out_type=jax.ShapeDtypeStruct((2, NPAD, D_H), _f32),
        mesh=_mesh,
        scratch_types=[
            pltpu.VMEM((MAXROWS, CHUNK), jnp.int32),
            pltpu.VMEM((CHUNK, D_H), _f32),
            pltpu.VMEM_SHARED((NPAD, D_H), _f32),
            pltpu.SemaphoreType.DMA,
        ],
        compiler_params=_SC_PARAMS,
    )
    return k(eidx3, zeros_n16, ones_r)


def _agg_body(y_hbm, eidx_hbm, zeros_hbm, out_hbm,
              isrc_v, idst_v, rows_v, acc_sh, y_sh, sem_g, sem_s):
    c = lax.axis_index("c")
    s = lax.axis_index("s")
    wid = s * 2 + c
    start, n, sstart, delta = _tile_range(wid)
    pltpu.sync_copy(eidx_hbm.at[0, pl.ds(sstart, MAXROWS)], isrc_v)
    pltpu.sync_copy(eidx_hbm.at[1, pl.ds(sstart, MAXROWS)], idst_v)
    pltpu.sync_copy(zeros_hbm.at[pl.ds(s * STRIPE, STRIPE)],
                    acc_sh.at[pl.ds(s * STRIPE, STRIPE)])
    pltpu.sync_copy(y_hbm.at[pl.ds(s * STRIPE, STRIPE)],
                    y_sh.at[pl.ds(s * STRIPE, STRIPE)])
    plsc.subcore_barrier()

    @pl.loop(0, NGROUP)
    def _(g):
        for b in range(GROUP):
            @pl.when(g * GROUP + b < n)
            def _(b=b):
                j = g * GROUP + b
                pltpu.async_copy(y_sh.at[isrc_v.at[delta + j]],
                                 rows_v.at[pl.ds(b * CHUNK, CHUNK)], sem_g)
        for b in range(GROUP):
            @pl.when(g * GROUP + b < n)
            def _(b=b):
                j = g * GROUP + b
                pltpu.make_async_copy(y_sh.at[isrc_v.at[delta + j]],
                                      rows_v.at[pl.ds(b * CHUNK, CHUNK)],
                                      sem_g).wait()
        for b in range(GROUP):
            @pl.when(g * GROUP + b < n)
            def _(b=b):
                j = g * GROUP + b
                pltpu.async_copy(rows_v.at[pl.ds(b * CHUNK, CHUNK)],
                                 acc_sh.at[idst_v.at[delta + j]], sem_s,
                                 add=True)
        for b in range(GROUP):
            @pl.when(g * GROUP + b < n)
            def _(b=b):
                j = g * GROUP + b
                pltpu.make_async_copy(rows_v.at[pl.ds(b * CHUNK, CHUNK)],
                                      acc_sh.at[idst_v.at[delta + j]],
                                      sem_s).wait()

    plsc.subcore_barrier()
    pltpu.sync_copy(acc_sh.at[pl.ds(s * STRIPE, STRIPE)],
                    out_hbm.at[c, pl.ds(s * STRIPE, STRIPE)])


def _agg_call(y, eidx3, zeros_n16):
    k = pl.kernel(
        _agg_body,
        out_type=jax.ShapeDtypeStruct((2, NPAD, D_H), _f32),
        mesh=_mesh,
        scratch_types=[
            pltpu.VMEM((MAXROWS, CHUNK), jnp.int32),
            pltpu.VMEM((MAXROWS, CHUNK), jnp.int32),
            pltpu.VMEM((GROUP * CHUNK, D_H), _f32),
            pltpu.VMEM_SHARED((NPAD, D_H), _f32),
            pltpu.VMEM_SHARED((NPAD, D_H), _f32),
            pltpu.SemaphoreType.DMA,
            pltpu.SemaphoreType.DMA,
        ],
        compiler_params=_SC_PARAMS,
    )
    return k(y, eidx3, zeros_n16)


def _gath_body(emb_hbm, eidx_hbm, gs_hbm, gd_hbm,
               isrc_v, idst_v, bs_v, bd_v, emb_sh, sem_g):
    c = lax.axis_index("c")
    s = lax.axis_index("s")
    wid = s * 2 + c
    start, n, sstart, delta = _tile_range(wid)
    pltpu.sync_copy(eidx_hbm.at[0, pl.ds(sstart, MAXROWS)], isrc_v)
    pltpu.sync_copy(eidx_hbm.at[1, pl.ds(sstart, MAXROWS)], idst_v)
    pltpu.sync_copy(emb_hbm.at[pl.ds(s * STRIPE, STRIPE)],
                    emb_sh.at[pl.ds(s * STRIPE, STRIPE)])
    plsc.subcore_barrier()

    @pl.loop(0, NGROUP)
    def _(g):
        for b in range(GROUP):
            @pl.when(g * GROUP + b < n)
            def _(b=b):
                j = g * GROUP + b
                pltpu.async_copy(emb_sh.at[isrc_v.at[delta + j]],
                                 bs_v.at[pl.ds(b * CHUNK, CHUNK)], sem_g)
                pltpu.async_copy(emb_sh.at[idst_v.at[delta + j]],
                                 bd_v.at[pl.ds(b * CHUNK, CHUNK)], sem_g)
        for b in range(GROUP):
            @pl.when(g * GROUP + b < n)
            def _(b=b):
                j = g * GROUP + b
                pltpu.make_async_copy(emb_sh.at[isrc_v.at[delta + j]],
                                      bs_v.at[pl.ds(b * CHUNK, CHUNK)],
                                      sem_g).wait()
                pltpu.make_async_copy(emb_sh.at[idst_v.at[delta + j]],
                                      bd_v.at[pl.ds(b * CHUNK, CHUNK)],
                                      sem_g).wait()

        @pl.when(g * GROUP + GROUP <= n)
        def _():
            base = (start + g * GROUP) * CHUNK
            pltpu.sync_copy(bs_v, gs_hbm.at[pl.ds(base, GROUP * CHUNK)])
            pltpu.sync_copy(bd_v, gd_hbm.at[pl.ds(base, GROUP * CHUNK)])

        @pl.when(jnp.logical_and(g * GROUP < n, g * GROUP + GROUP > n))
        def _():
            for b in range(GROUP):
                @pl.when(g * GROUP + b < n)
                def _(b=b):
                    j = g * GROUP + b
                    base = (start + j) * CHUNK
                    pltpu.sync_copy(bs_v.at[pl.ds(b * CHUNK, CHUNK)],
                                    gs_hbm.at[pl.ds(base, CHUNK)])
                    pltpu.sync_copy(bd_v.at[pl.ds(b * CHUNK, CHUNK)],
                                    gd_hbm.at[pl.ds(base, CHUNK)])


def _gath_call(emb, eidx3):
    k = pl.kernel(
        _gath_body,
        out_type=(jax.ShapeDtypeStruct((E, D_H), _f32),
                  jax.ShapeDtypeStruct((E, D_H), _f32)),
        mesh=_mesh,
        scratch_types=[
            pltpu.VMEM((MAXROWS, CHUNK), jnp.int32),
            pltpu.VMEM((MAXROWS, CHUNK), jnp.int32),
            pltpu.VMEM((GROUP * CHUNK, D_H), _f32),
            pltpu.VMEM((GROUP * CHUNK, D_H), _f32),
            pltpu.VMEM_SHARED((NPAD, D_H), _f32),
            pltpu.SemaphoreType.DMA,
        ],
        compiler_params=_SC_PARAMS,
    )
    return k(emb, eidx3)


_BR = 256
_NB = G4 // _BR


def _matvec(w, v):
    return lax.dot_general(w, v, (((1,), (0,)), ((), ())),
                           preferred_element_type=_f32)


def _lstm_body(wih_ref, whh_ref, cur0_ref, bih_ref, bhh_ref, hout_ref,
               gates_ref, wsum_ref, h_ref, c_ref):
    t = pl.program_id(0)
    nb = pl.program_id(1)
    rows = pl.ds(nb * _BR, _BR)

    @pl.when(jnp.logical_and(t == 0, nb == 0))
    def _():
        h_ref[...] = jnp.zeros_like(h_ref)
        c_ref[...] = jnp.zeros_like(c_ref)

    @pl.when(t == 0)
    def _():
        wih = wih_ref[...]
        whh = whh_ref[...]
        wsum_ref[rows, :] = (wih + whh).astype(_bf16)
        gates_ref[rows, :] = _matvec(wih.astype(_bf16),
                                     cur0_ref[...].astype(_bf16))

    @pl.when(t > 0)
    def _():
        w = wsum_ref[rows, :]
        hv = h_ref[...].astype(_bf16)
        gates_ref[rows, :] = _matvec(w, hv)

    @pl.when(nb == _NB - 1)
    def _():
        gf = gates_ref[...] + bih_ref[...] + bhh_ref[...]
        i = jax.nn.sigmoid(gf[0:FLAT])
        f = jax.nn.sigmoid(gf[FLAT:2 * FLAT])
        gg = jnp.tanh(gf[2 * FLAT:3 * FLAT])
        o = jax.nn.sigmoid(gf[3 * FLAT:4 * FLAT])
        c_new = f * c_ref[...] + i * gg
        h_new = o * jnp.tanh(c_new)
        c_ref[...] = c_new
        h_ref[...] = h_new

        @pl.when(t == T - 1)
        def _():
            hout_ref[...] = h_new


def _lstm_call(W_ih, W_hh, cur0, b_ih, b_hh):
    wmap = lambda t, nb: (jnp.where(t == 0, nb, _NB - 1), 0)
    return pl.pallas_call(
        _lstm_body,
        grid=(T, _NB),
        in_specs=[
            pl.BlockSpec((_BR, FLAT), wmap),
            pl.BlockSpec((_BR, FLAT), wmap),
            pl.BlockSpec((FLAT, 1), lambda t, nb: (0, 0)),
            pl.BlockSpec((G4, 1), lambda t, nb: (0, 0)),
            pl.BlockSpec((G4, 1), lambda t, nb: (0, 0)),
        ],
        out_specs=pl.BlockSpec((FLAT, 1), lambda t, nb: (0, 0)),
        out_shape=jax.ShapeDtypeStruct((FLAT, 1), _f32),
        scratch_shapes=[
            pltpu.VMEM((G4, 1), _f32),
            pltpu.VMEM((G4, FLAT), _bf16),
            pltpu.VMEM((FLAT, 1), _f32),
            pltpu.VMEM((FLAT, 1), _f32),
        ],
        compiler_params=pltpu.CompilerParams(
            dimension_semantics=("arbitrary", "arbitrary")),
    )(W_ih, W_hh, cur0, b_ih, b_hh)


_BN = 2048


def _ydis_body(x_ref, w_ref, da_ref, db_ref, y_ref, dis_ref):
    deg = da_ref[...][:, 0:1] + db_ref[...][:, 0:1] + 1.0
    dis = lax.rsqrt(deg)
    xw = jnp.dot(x_ref[...], w_ref[...], preferred_element_type=_f32)
    dis_ref[...] = dis
    y_ref[...] = dis * xw


def _ydis_call(x2p, w_fin, degA, degB):
    return pl.pallas_call(
        _ydis_body,
        grid=(NPAD // _BN,),
        in_specs=[
            pl.BlockSpec((_BN, D_IN), lambda i: (i, 0)),
            pl.BlockSpec((D_IN, D_H), lambda i: (0, 0)),
            pl.BlockSpec((_BN, D_H), lambda i: (i, 0)),
            pl.BlockSpec((_BN, D_H), lambda i: (i, 0)),
        ],
        out_specs=[
            pl.BlockSpec((_BN, D_H), lambda i: (i, 0)),
            pl.BlockSpec((_BN, 1), lambda i: (i, 0)),
        ],
        out_shape=[
            jax.ShapeDtypeStruct((NPAD, D_H), _f32),
            jax.ShapeDtypeStruct((NPAD, 1), _f32),
        ],
    )(x2p, w_fin, degA, degB)


def _emb_body(sa_ref, sb_ref, y_ref, dis_ref, emb_ref):
    tot = sa_ref[...] + sb_ref[...] + y_ref[...]
    emb_ref[...] = jnp.maximum(dis_ref[...] * tot, 0.0)


def _emb_call(sA, sB, y, dis):
    return pl.pallas_call(
        _emb_body,
        grid=(NPAD // _BN,),
        in_specs=[
            pl.BlockSpec((_BN, D_H), lambda i: (i, 0)),
            pl.BlockSpec((_BN, D_H), lambda i: (i, 0)),
            pl.BlockSpec((_BN, D_H), lambda i: (i, 0)),
            pl.BlockSpec((_BN, 1), lambda i: (i, 0)),
        ],
        out_specs=pl.BlockSpec((_BN, D_H), lambda i: (i, 0)),
        out_shape=jax.ShapeDtypeStruct((NPAD, D_H), _f32),
    )(sA, sB, y, dis)


_BE = 8192
_BR8 = _BE // 8
_NLANE = 8 * D_H


def _mlp_body(gs_ref, gd_ref, at_ref, bd1_ref, b1_ref, bd2_ref, b2_ref,
              out_ref):
    dot = lambda a, b: jnp.dot(a, b, preferred_element_type=_f32)
    bd1 = bd1_ref[...]
    hid = (dot(gs_ref[...], bd1[0:_NLANE]) + dot(gd_ref[...], bd1[_NLANE:2 * _NLANE])
           + dot(at_ref[...], bd1[2 * _NLANE:3 * _NLANE]) + b1_ref[...])
    hid = jnp.maximum(hid, 0.0)
    out_ref[...] = dot(hid, bd2_ref[...]) + b2_ref[...]


def _mlp_call(gs8, gd8, at8, bd1, b1p, bd2, b2):
    return pl.pallas_call(
        _mlp_body,
        grid=(pl.cdiv(E, _BE),),
        in_specs=[
            pl.BlockSpec((_BR8, _NLANE), lambda i: (i, 0)),
            pl.BlockSpec((_BR8, _NLANE), lambda i: (i, 0)),
            pl.BlockSpec((_BR8, _NLANE), lambda i: (i, 0)),
            pl.BlockSpec((3 * _NLANE, _NLANE), lambda i: (0, 0)),
            pl.BlockSpec((1, _NLANE), lambda i: (0, 0)),
            pl.BlockSpec((_NLANE, 8), lambda i: (0, 0)),
            pl.BlockSpec((1, 1), lambda i: (0, 0)),
        ],
        out_specs=pl.BlockSpec((_BR8, 8), lambda i: (i, 0)),
        out_shape=jax.ShapeDtypeStruct((E // 8, 8), _f32),
    )(gs8, gd8, at8, bd1, b1p, bd2, b2)


def kernel(x, edge_index, edge_attr, initial_weights, W_ih, W_hh, b_ih, b_hh,
           mlp_W1, mlp_b1, mlp_W2, mlp_b2):
    eidx3 = edge_index.reshape(2, NCHUNKS, CHUNK)
    at8 = edge_attr.reshape(E // 8, 8 * D_H)

    zeros_n16 = jnp.zeros((NPAD, D_H), _f32)
    ones_r = jnp.ones((CHUNK, D_H), _f32)

    deg2 = _deg_call(eidx3, zeros_n16, ones_r)
    h3 = _lstm_call(W_ih, W_hh, initial_weights.reshape(FLAT, 1),
                    b_ih.reshape(G4, 1), b_hh.reshape(G4, 1))
    w_fin = h3.reshape(D_IN, D_H)

    x2p = jnp.pad(x[T - 1], ((0, NPAD - N), (0, 0)))
    y, dis = _ydis_call(x2p, w_fin, deg2[0], deg2[1])

    s2 = _agg_call(y, eidx3, zeros_n16)
    emb = _emb_call(s2[0], s2[1], y, dis)
    gs, gd = _gath_call(emb, eidx3)

    eye8 = jnp.eye(8, dtype=_f32)
    bd1 = jnp.concatenate([jnp.kron(eye8, mlp_W1[k * D_H:(k + 1) * D_H])
                           for k in range(3)], axis=0)
    bd2 = jnp.kron(eye8, mlp_W2)
    b1p = jnp.tile(mlp_b1, 8).reshape(1, 8 * D_H)
    logits = _mlp_call(gs.reshape(E // 8, 8 * D_H),
                       gd.reshape(E // 8, 8 * D_H), at8,
                       bd1, b1p, bd2, mlp_b2.reshape(1, 1))
    return logits.reshape(E)
